# Initial kernel scaffold; baseline (speedup 1.0000x reference)
import jax
import jax.numpy as jnp
from jax.experimental import pallas as pl


def kernel(grid, edge_index, Wf1, bf1, Wf2, bf2, Wmap, bmap, Wl, bl, Wc1, bc1, Wc2, bc2, Wc3, bc3, Wcb1, bcb1, Wcb2, bcb2, Wt1, bt1, Wt2, bt2):
    raise NotImplementedError("write your pallas kernel here")


# speedup vs baseline: 1.4519x; 1.4519x over previous
import functools

import jax
import jax.numpy as jnp
import numpy as np
from jax import lax
from jax.experimental import pallas as pl
from jax.experimental.pallas import tpu as pltpu
from jax.experimental.pallas import tpu_sc as plsc

N = 102400
E = 408320
EP = 409600
FEAT = 32
STALK = 8
CH = 4
NPAT = 8
HALF = N // 2
SR = HALF + 128
ROWS_W = 2 * EP // 32
CB = 2560
KB = CB // 128
CBS = 512
KBS = CBS // 128
EB = 1280
NB = 1024

def _gather_body(x_hbm, ei_hbm, out_hbm, idx2d, rows, sem):
    c = lax.axis_index("c")
    s = lax.axis_index("s")
    w = s * 2 + c
    h = w // 16
    t = w % 16

    def outer(o, carry):
        base = t * ROWS_W + o * CB
        cps = [
            pltpu.async_copy(ei_hbm.at[h, pl.ds(base + j * 128, 128)],
                             idx2d.at[j], sem)
            for j in range(KB)
        ]
        for cp in cps:
            cp.wait()
        gs = [
            pltpu.async_copy(x_hbm.at[idx2d.at[j]],
                             rows.at[pl.ds(j * 128, 128), :], sem)
            for j in range(KB)
        ]
        for g in gs:
            g.wait()
        pltpu.sync_copy(rows, out_hbm.at[pl.ds(h * EP + base, CB), :])
        return carry

    lax.fori_loop(0, ROWS_W // CB, outer, 0)


_sc_cache = {}


def _get_sc_kernels():
    if "g" not in _sc_cache:
        mesh = plsc.VectorSubcoreMesh(core_axis_name="c",
                                      subcore_axis_name="s")
        params = pltpu.CompilerParams(use_tc_tiling_on_sc=False)
        _sc_cache["g"] = functools.partial(
            pl.kernel,
            out_type=jax.ShapeDtypeStruct((2 * EP, FEAT), jnp.float32),
            mesh=mesh,
            compiler_params=params,
            scratch_types=[
                pltpu.VMEM((KB, 128), jnp.int32),
                pltpu.VMEM((CB, FEAT), jnp.float32),
                pltpu.SemaphoreType.DMA,
            ],
        )(_gather_body)
        _sc_cache["s"] = functools.partial(
            pl.kernel,
            out_type=jax.ShapeDtypeStruct((N, FEAT), jnp.float32),
            mesh=mesh,
            compiler_params=params,
            scratch_types=[
                pltpu.VMEM((CBS,), jnp.int32),
                pltpu.VMEM((KBS, 128), jnp.int32),
                pltpu.VMEM((CBS, FEAT), jnp.float32),
                pltpu.VMEM_SHARED((SR, FEAT), jnp.float32),
                pltpu.SemaphoreType.DMA,
            ],
        )(_scatter_body)
    return _sc_cache["g"], _sc_cache["s"]


def _scatter_body(msg_hbm, dst_hbm, z_hbm, out_hbm, didx, idx2d, msgv, acc,
                  sem):
    c = lax.axis_index("c")
    t = lax.axis_index("s")
    lo = c * HALF
    pltpu.sync_copy(z_hbm.at[pl.ds(t * (SR // 16), SR // 16)],
                    acc.at[pl.ds(t * (SR // 16), SR // 16)])
    plsc.subcore_barrier()

    def outer(o, carry):
        base = t * (EP // 16) + o * CBS
        cp1 = pltpu.async_copy(dst_hbm.at[pl.ds(base, CBS)], didx, sem)
        cp2 = pltpu.async_copy(msg_hbm.at[pl.ds(base, CBS), :], msgv, sem)
        cp1.wait()
        cp2.wait()

        def conv_row(r, carry2):
            def conv16(q, carry3):
                v = didx[pl.ds((r * 8 + q) * 16, 16)]
                inr = (v >= lo) & (v < lo + HALF)
                idx2d[r, pl.ds(q * 16, 16)] = jnp.where(inr, v - lo,
                                                        HALF + t)
                return carry3

            return lax.fori_loop(0, 8, conv16, carry2)

        lax.fori_loop(0, KBS, conv_row, 0)
        adds = [
            pltpu.async_copy(msgv.at[pl.ds(j * 128, 128), :],
                             acc.at[idx2d.at[j]], sem, add=True)
            for j in range(KBS)
        ]
        for a in adds:
            a.wait()
        return carry

    lax.fori_loop(0, EP // 16 // CBS, outer, 0)
    plsc.subcore_barrier()
    pltpu.sync_copy(acc.at[pl.ds(t * (HALF // 16), HALF // 16)],
                    out_hbm.at[pl.ds(c * HALF + t * (HALF // 16),
                                     HALF // 16)])


def _sigmoid(x):
    return 1.0 / (1.0 + jnp.exp(-x))


def _k1_body(g_ref, w1, b1, w2, b2, out_ref):
    h = jnp.maximum(g_ref[...] @ w1[...] + b1[...], 0.0)
    out_ref[...] = h @ w2[...] + b2[...]


def _k2_body(xs_ref, xd_ref, w1, w2, b, out_ref):
    xs = xs_ref[...]
    mr = jnp.tanh(xs @ w1[...] + xd_ref[...] @ w2[...] + b[...])
    parts = []
    for i in range(8):
        part = mr[:, 32 * i:32 * i + 32] * xs
        p16 = part[:, :16] + part[:, 16:]
        p8 = p16[:, :8] + p16[:, 8:]
        parts.append(p8[:, :4] + p8[:, 4:])
    out_ref[...] = jnp.concatenate(parts, axis=1)


def _k3_body(x_ref, a_ref, w, b, out_ref):
    out_ref[...] = jnp.maximum((x_ref[...] - a_ref[...]) @ w[...] + b[...],
                               0.0)


def _k4_body(x_ref, w1, b1, w2s, b2, w3, b3, wcb1, bcb1, wcb2, bcb2, wt1,
             bt1, wt2, bt2, out_ref, ind_ref, comb_ref):
    x = x_ref[...]
    h1 = jnp.maximum(x @ w1[...] + b1[...], 0.0)
    h2s = [
        jnp.maximum(h1[:, 64 * p:64 * p + 64] @ w2s[p]
                    + b2[:, 64 * p:64 * p + 64], 0.0)
        for p in range(NPAT)
    ]
    h2 = jnp.concatenate(h2s, axis=1)
    ind = _sigmoid(h2 @ w3[...] + b3[...])
    ind_ref[...] = ind
    cb = jnp.maximum(ind @ wcb1[...] + bcb1[...], 0.0) @ wcb2[...] + bcb2[...]
    comb_ref[...] = _sigmoid(cb)
    feat = jnp.concatenate([x, ind], axis=1)
    out_ref[...] = (jnp.maximum(feat @ wt1[...] + bt1[...], 0.0) @ wt2[...]
                    + bt2[...])


def _full(shape):
    return pl.BlockSpec(shape, lambda i: tuple(0 for _ in shape))


def _k1(grid_a, Wf1, bf1, Wf2, bf2):
    return pl.pallas_call(
        _k1_body,
        grid=(N // NB,),
        in_specs=[
            pl.BlockSpec((NB, 10), lambda i: (i, 0)),
            _full((10, FEAT)), _full((1, FEAT)),
            _full((FEAT, FEAT)), _full((1, FEAT)),
        ],
        out_specs=pl.BlockSpec((NB, FEAT), lambda i: (i, 0)),
        out_shape=jax.ShapeDtypeStruct((N, FEAT), jnp.float32),
    )(grid_a, Wf1, bf1[None], Wf2, bf2[None])


def _k2(xg, w1r, w2r, br):
    return pl.pallas_call(
        _k2_body,
        grid=(EP // EB,),
        in_specs=[
            pl.BlockSpec((EB, FEAT), lambda i: (i, 0)),
            pl.BlockSpec((EB, FEAT), lambda i: (i + EP // EB, 0)),
            _full((FEAT, 256)), _full((FEAT, 256)), _full((1, 256)),
        ],
        out_specs=pl.BlockSpec((EB, FEAT), lambda i: (i, 0)),
        out_shape=jax.ShapeDtypeStruct((EP, FEAT), jnp.float32),
    )(xg, xg, w1r, w2r, br)


def _k3(x, agg, w, b):
    return pl.pallas_call(
        _k3_body,
        grid=(N // NB,),
        in_specs=[
            pl.BlockSpec((NB, FEAT), lambda i: (i, 0)),
            pl.BlockSpec((NB, FEAT), lambda i: (i, 0)),
            _full((FEAT, FEAT)), _full((1, FEAT)),
        ],
        out_specs=pl.BlockSpec((NB, FEAT), lambda i: (i, 0)),
        out_shape=jax.ShapeDtypeStruct((N, FEAT), jnp.float32),
    )(x, agg, w, b)


def _k4(x, w1, b1, w2s, b2, w3, b3, wcb1, bcb1, wcb2, bcb2, wt1, bt1, wt2,
        bt2):
    return pl.pallas_call(
        _k4_body,
        grid=(N // NB,),
        in_specs=[
            pl.BlockSpec((NB, FEAT), lambda i: (i, 0)),
            _full((FEAT, 512)), _full((1, 512)),
            _full((NPAT, 64, 64)), _full((1, 512)),
            _full((512, NPAT)), _full((1, NPAT)),
            _full((NPAT, 64)), _full((1, 64)),
            _full((64, 1)), _full((1, 1)),
            _full((FEAT + NPAT, FEAT)), _full((1, FEAT)),
            _full((FEAT, 10)), _full((1, 10)),
        ],
        out_specs=[
            pl.BlockSpec((NB, 10), lambda i: (i, 0)),
            pl.BlockSpec((NB, NPAT), lambda i: (i, 0)),
            pl.BlockSpec((NB, 1), lambda i: (i, 0)),
        ],
        out_shape=[
            jax.ShapeDtypeStruct((N, 10), jnp.float32),
            jax.ShapeDtypeStruct((N, NPAT), jnp.float32),
            jax.ShapeDtypeStruct((N, 1), jnp.float32),
        ],
    )(x, w1, b1, w2s, b2, w3, b3, wcb1, bcb1, wcb2, bcb2, wt1, bt1, wt2, bt2)


_q = np.arange(256)
_PERM = np.asarray(8 * (_q // 32) + (_q % 32) // 4, dtype=np.int32)


def kernel(grid, edge_index, Wf1, bf1, Wf2, bf2, Wmap, bmap, Wl, bl, Wc1,
           bc1, Wc2, bc2, Wc3, bc3, Wcb1, bcb1, Wcb2, bcb2, Wt1, bt1, Wt2,
           bt2):
    pad = EP - E
    ei_g = jnp.concatenate(
        [edge_index, jnp.zeros((2, pad), jnp.int32)], axis=1)
    dst_s = jnp.concatenate(
        [edge_index[1], jnp.full((pad,), N, jnp.int32)])
    zrows = jnp.zeros((SR, FEAT), jnp.float32)

    gather_fn, scatter_fn = _get_sc_kernels()
    x = _k1(grid, Wf1, bf1, Wf2, bf2)
    for l in range(2):
        wr = Wmap[l][:, _PERM]
        br = bmap[l][_PERM][None]
        xg = gather_fn(x, ei_g)
        msg = _k2(xg, wr[:FEAT], wr[FEAT:], br)
        agg = scatter_fn(msg, dst_s, zrows)
        x = _k3(x, agg, Wl[l], bl[l][None])

    w1all = jnp.transpose(Wc1, (1, 0, 2)).reshape(FEAT, NPAT * 64)
    b1all = bc1.reshape(1, NPAT * 64)
    b2all = bc2.reshape(1, NPAT * 64)
    w3blk = jax.scipy.linalg.block_diag(*[Wc3[p] for p in range(NPAT)])
    b3all = bc3.reshape(1, NPAT)
    out, individual, combined = _k4(
        x, w1all, b1all, Wc2, b2all, w3blk, b3all, Wcb1, bcb1[None],
        Wcb2, bcb2[None], Wt1, bt1[None], Wt2, bt2[None])
    return out, individual, combined


# speedup vs baseline: 1.4686x; 1.0115x over previous
import functools

import jax
import jax.numpy as jnp
import numpy as np
from jax import lax
from jax.experimental import pallas as pl
from jax.experimental.pallas import tpu as pltpu
from jax.experimental.pallas import tpu_sc as plsc

N = 102400
E = 408320
EP = 409600
FEAT = 32
STALK = 8
CH = 4
NPAT = 8
HALF = N // 2
SR = HALF + 128
ROWS_W = 2 * EP // 32
CB = 2560
KB = CB // 128
CBS = 512
KBS = CBS // 128
EB = 1280
NB = 1024
NBL = 6400

def _gather_body(x_hbm, ei_hbm, out_hbm, idx2d, rows, sem):
    c = lax.axis_index("c")
    s = lax.axis_index("s")
    w = s * 2 + c
    h = w // 16
    t = w % 16

    def outer(o, carry):
        base = t * ROWS_W + o * CB
        cps = [
            pltpu.async_copy(ei_hbm.at[h, pl.ds(base + j * 128, 128)],
                             idx2d.at[j], sem)
            for j in range(KB)
        ]
        for cp in cps:
            cp.wait()
        gs = [
            pltpu.async_copy(x_hbm.at[idx2d.at[j]],
                             rows.at[pl.ds(j * 128, 128), :], sem)
            for j in range(KB)
        ]
        for g in gs:
            g.wait()
        pltpu.sync_copy(rows, out_hbm.at[pl.ds(h * EP + base, CB), :])
        return carry

    lax.fori_loop(0, ROWS_W // CB, outer, 0)


_sc_cache = {}


def _get_sc_kernels():
    if "g" not in _sc_cache:
        mesh = plsc.VectorSubcoreMesh(core_axis_name="c",
                                      subcore_axis_name="s")
        params = pltpu.CompilerParams(use_tc_tiling_on_sc=False)
        _sc_cache["g"] = functools.partial(
            pl.kernel,
            out_type=jax.ShapeDtypeStruct((2 * EP, FEAT), jnp.float32),
            mesh=mesh,
            compiler_params=params,
            scratch_types=[
                pltpu.VMEM((KB, 128), jnp.int32),
                pltpu.VMEM((CB, FEAT), jnp.float32),
                pltpu.SemaphoreType.DMA,
            ],
        )(_gather_body)
        _sc_cache["s"] = functools.partial(
            pl.kernel,
            out_type=jax.ShapeDtypeStruct((N, FEAT), jnp.float32),
            mesh=mesh,
            compiler_params=params,
            scratch_types=[
                pltpu.VMEM((CBS,), jnp.int32),
                pltpu.VMEM((KBS, 128), jnp.int32),
                pltpu.VMEM((CBS, FEAT), jnp.float32),
                pltpu.VMEM_SHARED((SR, FEAT), jnp.float32),
                pltpu.SemaphoreType.DMA,
            ],
        )(_scatter_body)
    return _sc_cache["g"], _sc_cache["s"]


def _scatter_body(msg_hbm, dst_hbm, z_hbm, out_hbm, didx, idx2d, msgv, acc,
                  sem):
    c = lax.axis_index("c")
    t = lax.axis_index("s")
    lo = c * HALF
    pltpu.sync_copy(z_hbm.at[pl.ds(t * (SR // 16), SR // 16)],
                    acc.at[pl.ds(t * (SR // 16), SR // 16)])
    plsc.subcore_barrier()

    def outer(o, carry):
        base = t * (EP // 16) + o * CBS
        cp1 = pltpu.async_copy(dst_hbm.at[pl.ds(base, CBS)], didx, sem)
        cp2 = pltpu.async_copy(msg_hbm.at[pl.ds(base, CBS), :], msgv, sem)
        cp1.wait()
        cp2.wait()

        def conv_row(r, carry2):
            def conv16(q, carry3):
                v = didx[pl.ds((r * 8 + q) * 16, 16)]
                inr = (v >= lo) & (v < lo + HALF)
                idx2d[r, pl.ds(q * 16, 16)] = jnp.where(inr, v - lo,
                                                        HALF + t)
                return carry3

            return lax.fori_loop(0, 8, conv16, carry2)

        lax.fori_loop(0, KBS, conv_row, 0)
        adds = [
            pltpu.async_copy(msgv.at[pl.ds(j * 128, 128), :],
                             acc.at[idx2d.at[j]], sem, add=True)
            for j in range(KBS)
        ]
        for a in adds:
            a.wait()
        return carry

    lax.fori_loop(0, EP // 16 // CBS, outer, 0)
    plsc.subcore_barrier()
    pltpu.sync_copy(acc.at[pl.ds(t * (HALF // 16), HALF // 16)],
                    out_hbm.at[pl.ds(c * HALF + t * (HALF // 16),
                                     HALF // 16)])


def _sigmoid(x):
    return 1.0 / (1.0 + jnp.exp(-x))


def _k1_body(g_ref, w1, b1, w2, b2, out_ref):
    h = jnp.maximum(g_ref[...] @ w1[...] + b1[...], 0.0)
    out_ref[...] = h @ w2[...] + b2[...]


def _k2_body(xs_ref, xd_ref, w1, w2, b, rep, out_ref):
    xs = xs_ref[...]
    mr = jnp.tanh(xs @ w1[...] + xd_ref[...] @ w2[...] + b[...])
    mrr = mr @ rep[...]
    parts = []
    for i in range(8):
        part = mrr[:, 32 * i:32 * i + 32] * xs
        p16 = part[:, :16] + part[:, 16:]
        p8 = p16[:, :8] + p16[:, 8:]
        parts.append(p8[:, :4] + p8[:, 4:])
    out_ref[...] = jnp.concatenate(parts, axis=1)


def _k3_body(x_ref, a_ref, w, b, out_ref):
    out_ref[...] = jnp.maximum((x_ref[...] - a_ref[...]) @ w[...] + b[...],
                               0.0)


def _k4_body(x_ref, w1, b1, w2s, b2, w3, b3, wcb1, bcb1, wcb2, bcb2, wt1,
             bt1, wt2, bt2, out_ref, ind_ref, comb_ref):
    x = x_ref[...]
    h1 = jnp.maximum(x @ w1[...] + b1[...], 0.0)
    h2s = [
        jnp.maximum(h1[:, 64 * p:64 * p + 64] @ w2s[p]
                    + b2[:, 64 * p:64 * p + 64], 0.0)
        for p in range(NPAT)
    ]
    h2 = jnp.concatenate(h2s, axis=1)
    ind = _sigmoid(h2 @ w3[...] + b3[...])
    ind_ref[...] = ind
    cb = jnp.maximum(ind @ wcb1[...] + bcb1[...], 0.0) @ wcb2[...] + bcb2[...]
    comb_ref[...] = _sigmoid(cb)
    feat = jnp.concatenate([x, ind], axis=1)
    out_ref[...] = (jnp.maximum(feat @ wt1[...] + bt1[...], 0.0) @ wt2[...]
                    + bt2[...])


def _full(shape):
    return pl.BlockSpec(shape, lambda i: tuple(0 for _ in shape))


def _k1(grid_a, Wf1, bf1, Wf2, bf2):
    return pl.pallas_call(
        _k1_body,
        grid=(N // NBL,),
        in_specs=[
            pl.BlockSpec((NBL, 10), lambda i: (i, 0)),
            _full((10, FEAT)), _full((1, FEAT)),
            _full((FEAT, FEAT)), _full((1, FEAT)),
        ],
        out_specs=pl.BlockSpec((NBL, FEAT), lambda i: (i, 0)),
        out_shape=jax.ShapeDtypeStruct((N, FEAT), jnp.float32),
    )(grid_a, Wf1, bf1[None], Wf2, bf2[None])


def _k2(xg, w1, w2, b, rep):
    return pl.pallas_call(
        _k2_body,
        grid=(EP // EB,),
        in_specs=[
            pl.BlockSpec((EB, FEAT), lambda i: (i, 0)),
            pl.BlockSpec((EB, FEAT), lambda i: (i + EP // EB, 0)),
            _full((FEAT, 64)), _full((FEAT, 64)), _full((1, 64)),
            _full((64, 256)),
        ],
        out_specs=pl.BlockSpec((EB, FEAT), lambda i: (i, 0)),
        out_shape=jax.ShapeDtypeStruct((EP, FEAT), jnp.float32),
    )(xg, xg, w1, w2, b, rep)


def _k3(x, agg, w, b):
    return pl.pallas_call(
        _k3_body,
        grid=(N // NBL,),
        in_specs=[
            pl.BlockSpec((NBL, FEAT), lambda i: (i, 0)),
            pl.BlockSpec((NBL, FEAT), lambda i: (i, 0)),
            _full((FEAT, FEAT)), _full((1, FEAT)),
        ],
        out_specs=pl.BlockSpec((NBL, FEAT), lambda i: (i, 0)),
        out_shape=jax.ShapeDtypeStruct((N, FEAT), jnp.float32),
    )(x, agg, w, b)


def _k4(x, w1, b1, w2s, b2, w3, b3, wcb1, bcb1, wcb2, bcb2, wt1, bt1, wt2,
        bt2):
    return pl.pallas_call(
        _k4_body,
        grid=(N // NB,),
        in_specs=[
            pl.BlockSpec((NB, FEAT), lambda i: (i, 0)),
            _full((FEAT, 512)), _full((1, 512)),
            _full((NPAT, 64, 64)), _full((1, 512)),
            _full((512, NPAT)), _full((1, NPAT)),
            _full((NPAT, 64)), _full((1, 64)),
            _full((64, 1)), _full((1, 1)),
            _full((FEAT + NPAT, FEAT)), _full((1, FEAT)),
            _full((FEAT, 10)), _full((1, 10)),
        ],
        out_specs=[
            pl.BlockSpec((NB, 10), lambda i: (i, 0)),
            pl.BlockSpec((NB, NPAT), lambda i: (i, 0)),
            pl.BlockSpec((NB, 1), lambda i: (i, 0)),
        ],
        out_shape=[
            jax.ShapeDtypeStruct((N, 10), jnp.float32),
            jax.ShapeDtypeStruct((N, NPAT), jnp.float32),
            jax.ShapeDtypeStruct((N, 1), jnp.float32),
        ],
    )(x, w1, b1, w2s, b2, w3, b3, wcb1, bcb1, wcb2, bcb2, wt1, bt1, wt2, bt2)


_q = np.arange(256)
_PERM = np.asarray(8 * (_q // 32) + (_q % 32) // 4, dtype=np.int32)
_REP = np.zeros((64, 256), np.float32)
_REP[_PERM, _q] = 1.0


def kernel(grid, edge_index, Wf1, bf1, Wf2, bf2, Wmap, bmap, Wl, bl, Wc1,
           bc1, Wc2, bc2, Wc3, bc3, Wcb1, bcb1, Wcb2, bcb2, Wt1, bt1, Wt2,
           bt2):
    pad = EP - E
    ei_g = jnp.concatenate(
        [edge_index, jnp.zeros((2, pad), jnp.int32)], axis=1)
    dst_s = jnp.concatenate(
        [edge_index[1], jnp.full((pad,), N, jnp.int32)])
    zrows = jnp.zeros((SR, FEAT), jnp.float32)

    gather_fn, scatter_fn = _get_sc_kernels()
    rep = jnp.asarray(_REP)
    x = _k1(grid, Wf1, bf1, Wf2, bf2)
    for l in range(2):
        xg = gather_fn(x, ei_g)
        msg = _k2(xg, Wmap[l][:FEAT], Wmap[l][FEAT:], bmap[l][None], rep)
        agg = scatter_fn(msg, dst_s, zrows)
        x = _k3(x, agg, Wl[l], bl[l][None])

    w1all = jnp.transpose(Wc1, (1, 0, 2)).reshape(FEAT, NPAT * 64)
    b1all = bc1.reshape(1, NPAT * 64)
    b2all = bc2.reshape(1, NPAT * 64)
    w3blk = jax.scipy.linalg.block_diag(*[Wc3[p] for p in range(NPAT)])
    b3all = bc3.reshape(1, NPAT)
    out, individual, combined = _k4(
        x, w1all, b1all, Wc2, b2all, w3blk, b3all, Wcb1, bcb1[None],
        Wcb2, bcb2[None], Wt1, bt1[None], Wt2, bt2[None])
    return out, individual, combined


# speedup vs baseline: 3.0701x; 2.0905x over previous
import functools

import jax
import jax.numpy as jnp
import numpy as np
from jax import lax
from jax.experimental import pallas as pl
from jax.experimental.pallas import tpu as pltpu
from jax.experimental.pallas import tpu_sc as plsc

N = 102400
E = 408320
EP = 409600
FEAT = 32
STALK = 8
CH = 4
NPAT = 8
HALF = N // 2
SR = HALF + 128
ROWS_W = 2 * EP // 32
CB = 2560
KB = CB // 128
CBS = 512
KBS = CBS // 128
EB = 1280
NB = 1024
NBL = 6400

def _gather_body(x_hbm, ei_hbm, out_hbm, idx2d, rows, sem):
    c = lax.axis_index("c")
    s = lax.axis_index("s")
    w = s * 2 + c
    h = w // 16
    t = w % 16

    def outer(o, carry):
        base = t * ROWS_W + o * CB
        cps = [
            pltpu.async_copy(ei_hbm.at[h, pl.ds(base + j * 128, 128)],
                             idx2d.at[j], sem)
            for j in range(KB)
        ]
        for cp in cps:
            cp.wait()
        gs = [
            pltpu.async_copy(x_hbm.at[idx2d.at[j]],
                             rows.at[pl.ds(j * 128, 128), :], sem)
            for j in range(KB)
        ]
        for g in gs:
            g.wait()
        pltpu.sync_copy(rows, out_hbm.at[pl.ds(h * EP + base, CB), :])
        return carry

    lax.fori_loop(0, ROWS_W // CB, outer, 0)


_sc_cache = {}


def _get_sc_kernels():
    if "g" not in _sc_cache:
        mesh = plsc.VectorSubcoreMesh(core_axis_name="c",
                                      subcore_axis_name="s")
        params = pltpu.CompilerParams(use_tc_tiling_on_sc=False)
        _sc_cache["g"] = functools.partial(
            pl.kernel,
            out_type=jax.ShapeDtypeStruct((2 * EP, FEAT), jnp.float32),
            mesh=mesh,
            compiler_params=params,
            scratch_types=[
                pltpu.VMEM((KB, 128), jnp.int32),
                pltpu.VMEM((CB, FEAT), jnp.float32),
                pltpu.SemaphoreType.DMA,
            ],
        )(_gather_body)
        _sc_cache["s"] = functools.partial(
            pl.kernel,
            out_type=jax.ShapeDtypeStruct((N, FEAT), jnp.float32),
            mesh=mesh,
            compiler_params=params,
            scratch_types=[
                pltpu.VMEM((CBS,), jnp.int32),
                pltpu.VMEM((KBS, 128), jnp.int32),
                pltpu.VMEM((CBS, FEAT), jnp.float32),
                pltpu.VMEM_SHARED((SR, FEAT), jnp.float32),
                pltpu.SemaphoreType.DMA,
            ],
        )(_scatter_body)
    return _sc_cache["g"], _sc_cache["s"]


def _scatter_body(msg_hbm, dst_hbm, z_hbm, out_hbm, didx, idx2d, msgv, acc,
                  sem):
    c = lax.axis_index("c")
    t = lax.axis_index("s")
    lo = c * HALF
    pltpu.sync_copy(z_hbm.at[pl.ds(t * (SR // 16), SR // 16)],
                    acc.at[pl.ds(t * (SR // 16), SR // 16)])
    plsc.subcore_barrier()

    def outer(o, carry):
        base = t * (EP // 16) + o * CBS
        cp1 = pltpu.async_copy(dst_hbm.at[pl.ds(base, CBS)], didx, sem)
        cp2 = pltpu.async_copy(msg_hbm.at[pl.ds(base, CBS), :], msgv, sem)
        cp1.wait()
        cp2.wait()

        def conv_row(r, carry2):
            def conv16(q, carry3):
                v = didx[pl.ds((r * 8 + q) * 16, 16)]
                inr = (v >= lo) & (v < lo + HALF)
                idx2d[r, pl.ds(q * 16, 16)] = jnp.where(inr, v - lo,
                                                        HALF + t)
                return carry3

            return lax.fori_loop(0, 8, conv16, carry2)

        lax.fori_loop(0, KBS, conv_row, 0)
        adds = [
            pltpu.async_copy(msgv.at[pl.ds(j * 128, 128), :],
                             acc.at[idx2d.at[j]], sem, add=True)
            for j in range(KBS)
        ]
        for a in adds:
            a.wait()
        return carry

    lax.fori_loop(0, EP // 16 // CBS, outer, 0)
    plsc.subcore_barrier()
    pltpu.sync_copy(acc.at[pl.ds(t * (HALF // 16), HALF // 16)],
                    out_hbm.at[pl.ds(c * HALF + t * (HALF // 16),
                                     HALF // 16)])


def _sigmoid(x):
    return 1.0 / (1.0 + jnp.exp(-x))


def _k1_body(g_ref, w1, b1, w2, b2, out_ref):
    h = jnp.maximum(g_ref[...] @ w1[...] + b1[...], 0.0)
    out_ref[...] = h @ w2[...] + b2[...]


def _k2_body(xs_ref, xd_ref, w1, w2, b, rep, tile, sred, out_ref):
    xs = xs_ref[...]
    mr = jnp.tanh(xs @ w1[...] + xd_ref[...] @ w2[...] + b[...])
    mrr = mr @ rep[...]
    xsr = xs @ tile[...]
    out_ref[...] = (mrr * xsr) @ sred[...]


def _k3_body(x_ref, a_ref, w, b, out_ref):
    out_ref[...] = jnp.maximum((x_ref[...] - a_ref[...]) @ w[...] + b[...],
                               0.0)


def _k4_body(x_ref, w1, b1, w2s, b2, w3, b3, wcb1, bcb1, wcb2, bcb2, wt1,
             bt1, wt2, bt2, out_ref, ind_ref, comb_ref):
    x = x_ref[...]
    h1 = jnp.maximum(x @ w1[...] + b1[...], 0.0)
    h2s = [
        jnp.maximum(h1[:, 64 * p:64 * p + 64] @ w2s[p]
                    + b2[:, 64 * p:64 * p + 64], 0.0)
        for p in range(NPAT)
    ]
    h2 = jnp.concatenate(h2s, axis=1)
    ind = _sigmoid(h2 @ w3[...] + b3[...])
    ind_ref[...] = ind
    cb = jnp.maximum(ind @ wcb1[...] + bcb1[...], 0.0) @ wcb2[...] + bcb2[...]
    comb_ref[...] = _sigmoid(cb)
    feat = jnp.concatenate([x, ind], axis=1)
    out_ref[...] = (jnp.maximum(feat @ wt1[...] + bt1[...], 0.0) @ wt2[...]
                    + bt2[...])


def _full(shape):
    return pl.BlockSpec(shape, lambda i: tuple(0 for _ in shape))


def _k1(grid_a, Wf1, bf1, Wf2, bf2):
    return pl.pallas_call(
        _k1_body,
        grid=(N // NBL,),
        in_specs=[
            pl.BlockSpec((NBL, 10), lambda i: (i, 0)),
            _full((10, FEAT)), _full((1, FEAT)),
            _full((FEAT, FEAT)), _full((1, FEAT)),
        ],
        out_specs=pl.BlockSpec((NBL, FEAT), lambda i: (i, 0)),
        out_shape=jax.ShapeDtypeStruct((N, FEAT), jnp.float32),
    )(grid_a, Wf1, bf1[None], Wf2, bf2[None])


def _k2(xg, w1, w2, b, rep, tile, sred):
    return pl.pallas_call(
        _k2_body,
        grid=(EP // EB,),
        in_specs=[
            pl.BlockSpec((EB, FEAT), lambda i: (i, 0)),
            pl.BlockSpec((EB, FEAT), lambda i: (i + EP // EB, 0)),
            _full((FEAT, 64)), _full((FEAT, 64)), _full((1, 64)),
            _full((64, 256)), _full((FEAT, 256)), _full((256, FEAT)),
        ],
        out_specs=pl.BlockSpec((EB, FEAT), lambda i: (i, 0)),
        out_shape=jax.ShapeDtypeStruct((EP, FEAT), jnp.float32),
    )(xg, xg, w1, w2, b, rep, tile, sred)


def _k3(x, agg, w, b):
    return pl.pallas_call(
        _k3_body,
        grid=(N // NBL,),
        in_specs=[
            pl.BlockSpec((NBL, FEAT), lambda i: (i, 0)),
            pl.BlockSpec((NBL, FEAT), lambda i: (i, 0)),
            _full((FEAT, FEAT)), _full((1, FEAT)),
        ],
        out_specs=pl.BlockSpec((NBL, FEAT), lambda i: (i, 0)),
        out_shape=jax.ShapeDtypeStruct((N, FEAT), jnp.float32),
    )(x, agg, w, b)


def _k4(x, w1, b1, w2s, b2, w3, b3, wcb1, bcb1, wcb2, bcb2, wt1, bt1, wt2,
        bt2):
    return pl.pallas_call(
        _k4_body,
        grid=(N // NB,),
        in_specs=[
            pl.BlockSpec((NB, FEAT), lambda i: (i, 0)),
            _full((FEAT, 512)), _full((1, 512)),
            _full((NPAT, 64, 64)), _full((1, 512)),
            _full((512, NPAT)), _full((1, NPAT)),
            _full((NPAT, 64)), _full((1, 64)),
            _full((64, 1)), _full((1, 1)),
            _full((FEAT + NPAT, FEAT)), _full((1, FEAT)),
            _full((FEAT, 10)), _full((1, 10)),
        ],
        out_specs=[
            pl.BlockSpec((NB, 10), lambda i: (i, 0)),
            pl.BlockSpec((NB, NPAT), lambda i: (i, 0)),
            pl.BlockSpec((NB, 1), lambda i: (i, 0)),
        ],
        out_shape=[
            jax.ShapeDtypeStruct((N, 10), jnp.float32),
            jax.ShapeDtypeStruct((N, NPAT), jnp.float32),
            jax.ShapeDtypeStruct((N, 1), jnp.float32),
        ],
    )(x, w1, b1, w2s, b2, w3, b3, wcb1, bcb1, wcb2, bcb2, wt1, bt1, wt2, bt2)


_q = np.arange(256)
_PERM = np.asarray(8 * (_q // 32) + (_q % 32) // 4, dtype=np.int32)
_REP = np.zeros((64, 256), np.float32)
_REP[_PERM, _q] = 1.0
_TILE = np.zeros((32, 256), np.float32)
_TILE[_q % 32, _q] = 1.0
_SRED = np.zeros((256, 32), np.float32)
_SRED[_q, 4 * (_q // 32) + _q % 4] = 1.0


def kernel(grid, edge_index, Wf1, bf1, Wf2, bf2, Wmap, bmap, Wl, bl, Wc1,
           bc1, Wc2, bc2, Wc3, bc3, Wcb1, bcb1, Wcb2, bcb2, Wt1, bt1, Wt2,
           bt2):
    pad = EP - E
    ei_g = jnp.concatenate(
        [edge_index, jnp.zeros((2, pad), jnp.int32)], axis=1)
    dst_s = jnp.concatenate(
        [edge_index[1], jnp.full((pad,), N, jnp.int32)])
    zrows = jnp.zeros((SR, FEAT), jnp.float32)

    gather_fn, scatter_fn = _get_sc_kernels()
    rep = jnp.asarray(_REP)
    tile = jnp.asarray(_TILE)
    sred = jnp.asarray(_SRED)
    x = _k1(grid, Wf1, bf1, Wf2, bf2)
    for l in range(2):
        xg = gather_fn(x, ei_g)
        msg = _k2(xg, Wmap[l][:FEAT], Wmap[l][FEAT:], bmap[l][None], rep,
                  tile, sred)
        agg = scatter_fn(msg, dst_s, zrows)
        x = _k3(x, agg, Wl[l], bl[l][None])

    w1all = jnp.transpose(Wc1, (1, 0, 2)).reshape(FEAT, NPAT * 64)
    b1all = bc1.reshape(1, NPAT * 64)
    b2all = bc2.reshape(1, NPAT * 64)
    w3blk = jax.scipy.linalg.block_diag(*[Wc3[p] for p in range(NPAT)])
    b3all = bc3.reshape(1, NPAT)
    out, individual, combined = _k4(
        x, w1all, b1all, Wc2, b2all, w3blk, b3all, Wcb1, bcb1[None],
        Wcb2, bcb2[None], Wt1, bt1[None], Wt2, bt2[None])
    return out, individual, combined


# speedup vs baseline: 5.2704x; 1.7167x over previous
import functools

import jax
import jax.numpy as jnp
import numpy as np
from jax import lax
from jax.experimental import pallas as pl
from jax.experimental.pallas import tpu as pltpu
from jax.experimental.pallas import tpu_sc as plsc

N = 102400
E = 408320
EP = 409600
FEAT = 32
STALK = 8
CH = 4
NPAT = 8
HALF = N // 2
SR = HALF + 128
ROWS_W = 2 * EP // 32
CB = 2560
KB = CB // 128
CBS = 512
KBS = CBS // 128
EB = 1280
NB = 1024
NBL = 6400

def _gather_body(x_hbm, ei_hbm, out_hbm, idx2d, rows, sem):
    c = lax.axis_index("c")
    s = lax.axis_index("s")
    w = s * 2 + c
    h = w // 16
    t = w % 16

    def outer(o, carry):
        base = t * ROWS_W + o * CB
        cps = [
            pltpu.async_copy(ei_hbm.at[h, pl.ds(base + j * 128, 128)],
                             idx2d.at[j], sem)
            for j in range(KB)
        ]
        for cp in cps:
            cp.wait()
        gs = [
            pltpu.async_copy(x_hbm.at[idx2d.at[j]],
                             rows.at[pl.ds(j * 128, 128), :], sem)
            for j in range(KB)
        ]
        for g in gs:
            g.wait()
        pltpu.sync_copy(rows, out_hbm.at[pl.ds(h * EP + base, CB), :])
        return carry

    lax.fori_loop(0, ROWS_W // CB, outer, 0)


_sc_cache = {}


def _get_sc_kernels():
    if "g" not in _sc_cache:
        mesh = plsc.VectorSubcoreMesh(core_axis_name="c",
                                      subcore_axis_name="s")
        params = pltpu.CompilerParams(use_tc_tiling_on_sc=False)
        _sc_cache["g"] = functools.partial(
            pl.kernel,
            out_type=jax.ShapeDtypeStruct((2 * EP, FEAT), jnp.float32),
            mesh=mesh,
            compiler_params=params,
            scratch_types=[
                pltpu.VMEM((KB, 128), jnp.int32),
                pltpu.VMEM((CB, FEAT), jnp.float32),
                pltpu.SemaphoreType.DMA,
            ],
        )(_gather_body)
        _sc_cache["s"] = functools.partial(
            pl.kernel,
            out_type=jax.ShapeDtypeStruct((N, FEAT), jnp.float32),
            mesh=mesh,
            compiler_params=params,
            scratch_types=[
                pltpu.VMEM((CBS,), jnp.int32),
                pltpu.VMEM((KBS, 128), jnp.int32),
                pltpu.VMEM((CBS, FEAT), jnp.float32),
                pltpu.VMEM_SHARED((SR, FEAT), jnp.float32),
                pltpu.SemaphoreType.DMA,
            ],
        )(_scatter_body)
    return _sc_cache["g"], _sc_cache["s"]


def _scatter_body(msg_hbm, dst_hbm, z_hbm, out_hbm, didx, idx2d, msgv, acc,
                  sem):
    c = lax.axis_index("c")
    t = lax.axis_index("s")
    lo = c * HALF
    pltpu.sync_copy(z_hbm.at[pl.ds(t * (SR // 16), SR // 16)],
                    acc.at[pl.ds(t * (SR // 16), SR // 16)])
    plsc.subcore_barrier()

    def outer(o, carry):
        base = t * (EP // 16) + o * CBS
        cp1 = pltpu.async_copy(dst_hbm.at[pl.ds(base, CBS)], didx, sem)
        cp2 = pltpu.async_copy(msg_hbm.at[pl.ds(base, CBS), :], msgv, sem)
        cp1.wait()
        cp2.wait()

        def conv_row(r, carry2):
            def conv16(q, carry3):
                v = didx[pl.ds((r * 8 + q) * 16, 16)]
                inr = (v >= lo) & (v < lo + HALF)
                idx2d[r, pl.ds(q * 16, 16)] = jnp.where(inr, v - lo,
                                                        HALF + t)
                return carry3

            return lax.fori_loop(0, 8, conv16, carry2)

        lax.fori_loop(0, KBS, conv_row, 0)
        adds = [
            pltpu.async_copy(msgv.at[pl.ds(j * 128, 128), :],
                             acc.at[idx2d.at[j]], sem, add=True)
            for j in range(KBS)
        ]
        for a in adds:
            a.wait()
        return carry

    lax.fori_loop(0, EP // 16 // CBS, outer, 0)
    plsc.subcore_barrier()
    pltpu.sync_copy(acc.at[pl.ds(t * (HALF // 16), HALF // 16)],
                    out_hbm.at[pl.ds(c * HALF + t * (HALF // 16),
                                     HALF // 16)])


def _sigmoid(x):
    return 1.0 / (1.0 + jnp.exp(-x))


def _k1_body(g_ref, w1, b1, w2, b2, out_ref):
    h = jnp.maximum(g_ref[...] @ w1[...] + b1[...], 0.0)
    out_ref[...] = h @ w2[...] + b2[...]


def _bd4(w):
    return jax.scipy.linalg.block_diag(w, w, w, w)


def _k2_body(xs_ref, xd_ref, w1, w2, b, rep, tile, sred, out_ref):
    xs = xs_ref[...]
    mr = jnp.tanh(xs @ w1[...] + xd_ref[...] @ w2[...] + b[...])
    mrr = mr @ rep[...]
    xsr = xs @ tile[...]
    out_ref[...] = (mrr * xsr) @ sred[...]


def _k3_body(x_ref, a_ref, w, b, out_ref):
    out_ref[...] = jnp.maximum((x_ref[...] - a_ref[...]) @ w[...] + b[...],
                               0.0)


def _k4_body(x_ref, w1, b1, w2s, b2, w3, b3, wcb1, bcb1, wcb2, bcb2, wt1,
             bt1, wt2, bt2, out_ref, ind_ref, comb_ref):
    x = x_ref[...]
    h1 = jnp.maximum(x @ w1[...] + b1[...], 0.0)
    h2s = [
        jnp.maximum(h1[:, 64 * p:64 * p + 64] @ w2s[p]
                    + b2[:, 64 * p:64 * p + 64], 0.0)
        for p in range(NPAT)
    ]
    h2 = jnp.concatenate(h2s, axis=1)
    ind = _sigmoid(h2 @ w3[...] + b3[...])
    ind_ref[...] = ind
    cb = jnp.maximum(ind @ wcb1[...] + bcb1[...], 0.0) @ wcb2[...] + bcb2[...]
    comb_ref[...] = _sigmoid(cb)
    feat = jnp.concatenate([x, ind], axis=1)
    out_ref[...] = (jnp.maximum(feat @ wt1[...] + bt1[...], 0.0) @ wt2[...]
                    + bt2[...])


def _full(shape):
    return pl.BlockSpec(shape, lambda i: tuple(0 for _ in shape))


def _k1(grid_p, Wf1, bf1, Wf2, bf2):
    return pl.pallas_call(
        _k1_body,
        grid=(N // NBL,),
        in_specs=[
            pl.BlockSpec((NBL // 4, 40), lambda i: (i, 0)),
            _full((40, 128)), _full((1, 128)),
            _full((128, 128)), _full((1, 128)),
        ],
        out_specs=pl.BlockSpec((NBL // 4, 128), lambda i: (i, 0)),
        out_shape=jax.ShapeDtypeStruct((N // 4, 128), jnp.float32),
    )(grid_p, _bd4(Wf1), jnp.tile(bf1, 4)[None], _bd4(Wf2),
      jnp.tile(bf2, 4)[None])


def _k2(xg_p, w1, w2, b):
    eb4 = EB // 4
    return pl.pallas_call(
        _k2_body,
        grid=(EP // EB,),
        in_specs=[
            pl.BlockSpec((eb4, 128), lambda i: (i, 0)),
            pl.BlockSpec((eb4, 128), lambda i: (i + EP // EB, 0)),
            _full((128, 256)), _full((128, 256)), _full((1, 256)),
            _full((256, 1024)), _full((128, 1024)), _full((1024, 128)),
        ],
        out_specs=pl.BlockSpec((eb4, 128), lambda i: (i, 0)),
        out_shape=jax.ShapeDtypeStruct((EP // 4, 128), jnp.float32),
    )(xg_p, xg_p, _bd4(w1), _bd4(w2), jnp.tile(b[0], 4)[None],
      jnp.asarray(_BREP), jnp.asarray(_BTILE), jnp.asarray(_BSRED))


def _k3(x_p, agg_p, w, b):
    return pl.pallas_call(
        _k3_body,
        grid=(N // NBL,),
        in_specs=[
            pl.BlockSpec((NBL // 4, 128), lambda i: (i, 0)),
            pl.BlockSpec((NBL // 4, 128), lambda i: (i, 0)),
            _full((128, 128)), _full((1, 128)),
        ],
        out_specs=pl.BlockSpec((NBL // 4, 128), lambda i: (i, 0)),
        out_shape=jax.ShapeDtypeStruct((N // 4, 128), jnp.float32),
    )(x_p, agg_p, _bd4(w), jnp.tile(b[0], 4)[None])


def _k4(x, w1, b1, w2s, b2, w3, b3, wcb1, bcb1, wcb2, bcb2, wt1, bt1, wt2,
        bt2):
    return pl.pallas_call(
        _k4_body,
        grid=(N // NB,),
        in_specs=[
            pl.BlockSpec((NB, FEAT), lambda i: (i, 0)),
            _full((FEAT, 512)), _full((1, 512)),
            _full((NPAT, 64, 64)), _full((1, 512)),
            _full((512, NPAT)), _full((1, NPAT)),
            _full((NPAT, 64)), _full((1, 64)),
            _full((64, 1)), _full((1, 1)),
            _full((FEAT + NPAT, FEAT)), _full((1, FEAT)),
            _full((FEAT, 10)), _full((1, 10)),
        ],
        out_specs=[
            pl.BlockSpec((NB, 10), lambda i: (i, 0)),
            pl.BlockSpec((NB, NPAT), lambda i: (i, 0)),
            pl.BlockSpec((NB, 1), lambda i: (i, 0)),
        ],
        out_shape=[
            jax.ShapeDtypeStruct((N, 10), jnp.float32),
            jax.ShapeDtypeStruct((N, NPAT), jnp.float32),
            jax.ShapeDtypeStruct((N, 1), jnp.float32),
        ],
    )(x, w1, b1, w2s, b2, w3, b3, wcb1, bcb1, wcb2, bcb2, wt1, bt1, wt2, bt2)


_q = np.arange(256)
_PERM = np.asarray(8 * (_q // 32) + (_q % 32) // 4, dtype=np.int32)
_REP = np.zeros((64, 256), np.float32)
_REP[_PERM, _q] = 1.0
_TILE = np.zeros((32, 256), np.float32)
_TILE[_q % 32, _q] = 1.0
_SRED = np.zeros((256, 32), np.float32)
_SRED[_q, 4 * (_q // 32) + _q % 4] = 1.0


def _np_bd4(w):
    k, m = w.shape
    out = np.zeros((4 * k, 4 * m), np.float32)
    for u in range(4):
        out[u * k:(u + 1) * k, u * m:(u + 1) * m] = w
    return out


_BREP = _np_bd4(_REP)
_BTILE = _np_bd4(_TILE)
_BSRED = _np_bd4(_SRED)


def kernel(grid, edge_index, Wf1, bf1, Wf2, bf2, Wmap, bmap, Wl, bl, Wc1,
           bc1, Wc2, bc2, Wc3, bc3, Wcb1, bcb1, Wcb2, bcb2, Wt1, bt1, Wt2,
           bt2):
    pad = EP - E
    ei_g = jnp.concatenate(
        [edge_index, jnp.zeros((2, pad), jnp.int32)], axis=1)
    dst_s = jnp.concatenate(
        [edge_index[1], jnp.full((pad,), N, jnp.int32)])
    zrows = jnp.zeros((SR, FEAT), jnp.float32)

    gather_fn, scatter_fn = _get_sc_kernels()
    xp = _k1(grid.reshape(N // 4, 40), Wf1, bf1, Wf2, bf2)
    for l in range(2):
        xg = gather_fn(xp.reshape(N, FEAT), ei_g)
        msg_p = _k2(xg.reshape(2 * EP // 4, 128), Wmap[l][:FEAT],
                    Wmap[l][FEAT:], bmap[l][None])
        agg = scatter_fn(msg_p.reshape(EP, FEAT), dst_s, zrows)
        xp = _k3(xp, agg.reshape(N // 4, 128), Wl[l], bl[l][None])
    x = xp.reshape(N, FEAT)

    w1all = jnp.transpose(Wc1, (1, 0, 2)).reshape(FEAT, NPAT * 64)
    b1all = bc1.reshape(1, NPAT * 64)
    b2all = bc2.reshape(1, NPAT * 64)
    w3blk = jax.scipy.linalg.block_diag(*[Wc3[p] for p in range(NPAT)])
    b3all = bc3.reshape(1, NPAT)
    out, individual, combined = _k4(
        x, w1all, b1all, Wc2, b2all, w3blk, b3all, Wcb1, bcb1[None],
        Wcb2, bcb2[None], Wt1, bt1[None], Wt2, bt2[None])
    return out, individual, combined


# speedup vs baseline: 5.7802x; 1.0967x over previous
import functools

import jax
import jax.numpy as jnp
import numpy as np
from jax import lax
from jax.experimental import pallas as pl
from jax.experimental.pallas import tpu as pltpu
from jax.experimental.pallas import tpu_sc as plsc

N = 102400
E = 408320
EP = 409600
FEAT = 32
STALK = 8
CH = 4
NPAT = 8
HALF = N // 2
SR = HALF + 128
ROWS_W = 2 * EP // 32
CB = 2560
KB = CB // 128
CBS = 512
KBS = CBS // 128
EB = 2560
NB = 1024
NBL = 6400

def _gather_body(x_hbm, ei_hbm, out_hbm, idx1, rows, sem):
    c = lax.axis_index("c")
    s = lax.axis_index("s")
    w = s * 2 + c
    h = w // 16
    t = w % 16

    def outer(o, carry):
        base = t * ROWS_W + o * CB
        pltpu.async_copy(ei_hbm.at[h, pl.ds(base, CB)], idx1, sem).wait()
        gs = [
            pltpu.async_copy(x_hbm.at[idx1.at[pl.ds(j * 128, 128)]],
                             rows.at[pl.ds(j * 128, 128), :], sem)
            for j in range(KB)
        ]
        for g in gs:
            g.wait()
        pltpu.sync_copy(rows, out_hbm.at[pl.ds(h * EP + base, CB), :])
        return carry

    lax.fori_loop(0, ROWS_W // CB, outer, 0)


_sc_cache = {}


def _get_sc_kernels():
    if "g" not in _sc_cache:
        mesh = plsc.VectorSubcoreMesh(core_axis_name="c",
                                      subcore_axis_name="s")
        params = pltpu.CompilerParams(use_tc_tiling_on_sc=False)
        _sc_cache["g"] = functools.partial(
            pl.kernel,
            out_type=jax.ShapeDtypeStruct((2 * EP, FEAT), jnp.float32),
            mesh=mesh,
            compiler_params=params,
            scratch_types=[
                pltpu.VMEM((CB,), jnp.int32),
                pltpu.VMEM((CB, FEAT), jnp.float32),
                pltpu.SemaphoreType.DMA,
            ],
        )(_gather_body)
        _sc_cache["s"] = functools.partial(
            pl.kernel,
            out_type=jax.ShapeDtypeStruct((N, FEAT), jnp.float32),
            mesh=mesh,
            compiler_params=params,
            scratch_types=[
                pltpu.VMEM((CBS,), jnp.int32),
                pltpu.VMEM((KBS, 128), jnp.int32),
                pltpu.VMEM((CBS, FEAT), jnp.float32),
                pltpu.VMEM_SHARED((SR, FEAT), jnp.float32),
                pltpu.SemaphoreType.DMA,
            ],
        )(_scatter_body)
    return _sc_cache["g"], _sc_cache["s"]


def _scatter_body(msg_hbm, dst_hbm, z_hbm, out_hbm, didx, idx2d, msgv, acc,
                  sem):
    c = lax.axis_index("c")
    t = lax.axis_index("s")
    lo = c * HALF
    pltpu.sync_copy(z_hbm.at[pl.ds(t * (SR // 16), SR // 16)],
                    acc.at[pl.ds(t * (SR // 16), SR // 16)])
    plsc.subcore_barrier()

    def outer(o, carry):
        base = t * (EP // 16) + o * CBS
        cp1 = pltpu.async_copy(dst_hbm.at[pl.ds(base, CBS)], didx, sem)
        cp2 = pltpu.async_copy(msg_hbm.at[pl.ds(base, CBS), :], msgv, sem)
        cp1.wait()
        cp2.wait()

        def conv_row(r, carry2):
            def conv16(q, carry3):
                v = didx[pl.ds((r * 8 + q) * 16, 16)]
                inr = (v >= lo) & (v < lo + HALF)
                idx2d[r, pl.ds(q * 16, 16)] = jnp.where(inr, v - lo,
                                                        HALF + t)
                return carry3

            return lax.fori_loop(0, 8, conv16, carry2)

        lax.fori_loop(0, KBS, conv_row, 0)
        adds = [
            pltpu.async_copy(msgv.at[pl.ds(j * 128, 128), :],
                             acc.at[idx2d.at[j]], sem, add=True)
            for j in range(KBS)
        ]
        for a in adds:
            a.wait()
        return carry

    lax.fori_loop(0, EP // 16 // CBS, outer, 0)
    plsc.subcore_barrier()
    pltpu.sync_copy(acc.at[pl.ds(t * (HALF // 16), HALF // 16)],
                    out_hbm.at[pl.ds(c * HALF + t * (HALF // 16),
                                     HALF // 16)])


def _sigmoid(x):
    return 1.0 / (1.0 + jnp.exp(-x))


def _k1_body(g_ref, w1, b1, w2, b2, out_ref):
    h = jnp.maximum(g_ref[...] @ w1[...] + b1[...], 0.0)
    out_ref[...] = h @ w2[...] + b2[...]


def _bd4(w):
    return jax.scipy.linalg.block_diag(w, w, w, w)


def _k2_body(xs_ref, xd_ref, w1, w2, b, rep, tile, sred, out_ref):
    xs = xs_ref[...]
    mr = jnp.tanh(xs @ w1[...] + xd_ref[...] @ w2[...] + b[...])
    mrr = mr @ rep[...]
    xsr = xs @ tile[...]
    out_ref[...] = (mrr * xsr) @ sred[...]


def _k3_body(x_ref, a_ref, w, b, out_ref):
    out_ref[...] = jnp.maximum((x_ref[...] - a_ref[...]) @ w[...] + b[...],
                               0.0)


def _k4_body(x_ref, w1, b1, w2s, b2, w3, b3, wcb1, bcb1, wcb2, bcb2, wt1,
             bt1, wt2, bt2, out_ref, ind_ref, comb_ref):
    x = x_ref[...]
    h1 = jnp.maximum(x @ w1[...] + b1[...], 0.0)
    h2s = [
        jnp.maximum(h1[:, 64 * p:64 * p + 64] @ w2s[p]
                    + b2[:, 64 * p:64 * p + 64], 0.0)
        for p in range(NPAT)
    ]
    h2 = jnp.concatenate(h2s, axis=1)
    ind = _sigmoid(h2 @ w3[...] + b3[...])
    ind_ref[...] = ind
    cb = jnp.maximum(ind @ wcb1[...] + bcb1[...], 0.0) @ wcb2[...] + bcb2[...]
    comb_ref[...] = _sigmoid(cb)
    feat = jnp.concatenate([x, ind], axis=1)
    out_ref[...] = (jnp.maximum(feat @ wt1[...] + bt1[...], 0.0) @ wt2[...]
                    + bt2[...])


def _full(shape):
    return pl.BlockSpec(shape, lambda i: tuple(0 for _ in shape))


def _k1(grid_p, Wf1, bf1, Wf2, bf2):
    return pl.pallas_call(
        _k1_body,
        grid=(N // NBL,),
        in_specs=[
            pl.BlockSpec((NBL // 4, 40), lambda i: (i, 0)),
            _full((40, 128)), _full((1, 128)),
            _full((128, 128)), _full((1, 128)),
        ],
        out_specs=pl.BlockSpec((NBL // 4, 128), lambda i: (i, 0)),
        out_shape=jax.ShapeDtypeStruct((N // 4, 128), jnp.float32),
    )(grid_p, _bd4(Wf1), jnp.tile(bf1, 4)[None], _bd4(Wf2),
      jnp.tile(bf2, 4)[None])


def _k2(xg_p, w1, w2, b):
    eb4 = EB // 4
    return pl.pallas_call(
        _k2_body,
        grid=(EP // EB,),
        in_specs=[
            pl.BlockSpec((eb4, 128), lambda i: (i, 0)),
            pl.BlockSpec((eb4, 128), lambda i: (i + EP // EB, 0)),
            _full((128, 256)), _full((128, 256)), _full((1, 256)),
            _full((256, 1024)), _full((128, 1024)), _full((1024, 128)),
        ],
        out_specs=pl.BlockSpec((eb4, 128), lambda i: (i, 0)),
        out_shape=jax.ShapeDtypeStruct((EP // 4, 128), jnp.float32),
    )(xg_p, xg_p, _bd4(w1), _bd4(w2), jnp.tile(b[0], 4)[None],
      jnp.asarray(_BREP), jnp.asarray(_BTILE), jnp.asarray(_BSRED))


def _k3(x_p, agg_p, w, b):
    return pl.pallas_call(
        _k3_body,
        grid=(N // NBL,),
        in_specs=[
            pl.BlockSpec((NBL // 4, 128), lambda i: (i, 0)),
            pl.BlockSpec((NBL // 4, 128), lambda i: (i, 0)),
            _full((128, 128)), _full((1, 128)),
        ],
        out_specs=pl.BlockSpec((NBL // 4, 128), lambda i: (i, 0)),
        out_shape=jax.ShapeDtypeStruct((N // 4, 128), jnp.float32),
    )(x_p, agg_p, _bd4(w), jnp.tile(b[0], 4)[None])


def _k4(x, w1, b1, w2s, b2, w3, b3, wcb1, bcb1, wcb2, bcb2, wt1, bt1, wt2,
        bt2):
    return pl.pallas_call(
        _k4_body,
        grid=(N // NB,),
        in_specs=[
            pl.BlockSpec((NB, FEAT), lambda i: (i, 0)),
            _full((FEAT, 512)), _full((1, 512)),
            _full((NPAT, 64, 64)), _full((1, 512)),
            _full((512, NPAT)), _full((1, NPAT)),
            _full((NPAT, 64)), _full((1, 64)),
            _full((64, 1)), _full((1, 1)),
            _full((FEAT + NPAT, FEAT)), _full((1, FEAT)),
            _full((FEAT, 10)), _full((1, 10)),
        ],
        out_specs=[
            pl.BlockSpec((NB, 10), lambda i: (i, 0)),
            pl.BlockSpec((NB, NPAT), lambda i: (i, 0)),
            pl.BlockSpec((NB, 1), lambda i: (i, 0)),
        ],
        out_shape=[
            jax.ShapeDtypeStruct((N, 10), jnp.float32),
            jax.ShapeDtypeStruct((N, NPAT), jnp.float32),
            jax.ShapeDtypeStruct((N, 1), jnp.float32),
        ],
    )(x, w1, b1, w2s, b2, w3, b3, wcb1, bcb1, wcb2, bcb2, wt1, bt1, wt2, bt2)


_q = np.arange(256)
_PERM = np.asarray(8 * (_q // 32) + (_q % 32) // 4, dtype=np.int32)
_REP = np.zeros((64, 256), np.float32)
_REP[_PERM, _q] = 1.0
_TILE = np.zeros((32, 256), np.float32)
_TILE[_q % 32, _q] = 1.0
_SRED = np.zeros((256, 32), np.float32)
_SRED[_q, 4 * (_q // 32) + _q % 4] = 1.0


def _np_bd4(w):
    k, m = w.shape
    out = np.zeros((4 * k, 4 * m), np.float32)
    for u in range(4):
        out[u * k:(u + 1) * k, u * m:(u + 1) * m] = w
    return out


_BREP = _np_bd4(_REP)
_BTILE = _np_bd4(_TILE)
_BSRED = _np_bd4(_SRED)


def kernel(grid, edge_index, Wf1, bf1, Wf2, bf2, Wmap, bmap, Wl, bl, Wc1,
           bc1, Wc2, bc2, Wc3, bc3, Wcb1, bcb1, Wcb2, bcb2, Wt1, bt1, Wt2,
           bt2):
    pad = EP - E
    ei_g = jnp.concatenate(
        [edge_index, jnp.zeros((2, pad), jnp.int32)], axis=1)
    dst_s = jnp.concatenate(
        [edge_index[1], jnp.full((pad,), N, jnp.int32)])
    zrows = jnp.zeros((SR, FEAT), jnp.float32)

    gather_fn, scatter_fn = _get_sc_kernels()
    xp = _k1(grid.reshape(N // 4, 40), Wf1, bf1, Wf2, bf2)
    for l in range(2):
        xg = gather_fn(xp.reshape(N, FEAT), ei_g)
        msg_p = _k2(xg.reshape(2 * EP // 4, 128), Wmap[l][:FEAT],
                    Wmap[l][FEAT:], bmap[l][None])
        agg = scatter_fn(msg_p.reshape(EP, FEAT), dst_s, zrows)
        xp = _k3(xp, agg.reshape(N // 4, 128), Wl[l], bl[l][None])
    x = xp.reshape(N, FEAT)

    w1all = jnp.transpose(Wc1, (1, 0, 2)).reshape(FEAT, NPAT * 64)
    b1all = bc1.reshape(1, NPAT * 64)
    b2all = bc2.reshape(1, NPAT * 64)
    w3blk = jax.scipy.linalg.block_diag(*[Wc3[p] for p in range(NPAT)])
    b3all = bc3.reshape(1, NPAT)
    out, individual, combined = _k4(
        x, w1all, b1all, Wc2, b2all, w3blk, b3all, Wcb1, bcb1[None],
        Wcb2, bcb2[None], Wt1, bt1[None], Wt2, bt2[None])
    return out, individual, combined


# speedup vs baseline: 6.2417x; 1.0798x over previous
import functools

import jax
import jax.numpy as jnp
import numpy as np
from jax import lax
from jax.experimental import pallas as pl
from jax.experimental.pallas import tpu as pltpu
from jax.experimental.pallas import tpu_sc as plsc

N = 102400
E = 408320
EP = 409600
FEAT = 32
STALK = 8
CH = 4
NPAT = 8
HALF = N // 2
SR = HALF + 128
ROWS_W = 2 * EP // 32
CB = 2560
KB = CB // 128
CBS = 512
KBS = CBS // 128
EB = 2560
NB = 1024
NBL = 6400

def _gather_body(x_hbm, ei_hbm, out_hbm, idx1, rows, sem):
    c = lax.axis_index("c")
    s = lax.axis_index("s")
    w = s * 2 + c
    h = w // 16
    t = w % 16

    def outer(o, carry):
        base = t * ROWS_W + o * CB
        pltpu.async_copy(ei_hbm.at[h, pl.ds(base, CB)], idx1, sem).wait()
        gs = [
            pltpu.async_copy(x_hbm.at[idx1.at[pl.ds(j * 128, 128)]],
                             rows.at[pl.ds(j * 128, 128), :], sem)
            for j in range(KB)
        ]
        for g in gs:
            g.wait()
        pltpu.sync_copy(rows, out_hbm.at[pl.ds(h * EP + base, CB), :])
        return carry

    lax.fori_loop(0, ROWS_W // CB, outer, 0)


_sc_cache = {}


def _get_sc_kernels():
    if "g" not in _sc_cache:
        mesh = plsc.VectorSubcoreMesh(core_axis_name="c",
                                      subcore_axis_name="s")
        params = pltpu.CompilerParams(use_tc_tiling_on_sc=False)
        _sc_cache["g"] = functools.partial(
            pl.kernel,
            out_type=jax.ShapeDtypeStruct((2 * EP, FEAT), jnp.float32),
            mesh=mesh,
            compiler_params=params,
            scratch_types=[
                pltpu.VMEM((CB,), jnp.int32),
                pltpu.VMEM((CB, FEAT), jnp.float32),
                pltpu.SemaphoreType.DMA,
            ],
        )(_gather_body)
        _sc_cache["s"] = functools.partial(
            pl.kernel,
            out_type=jax.ShapeDtypeStruct((N, FEAT), jnp.float32),
            mesh=mesh,
            compiler_params=params,
            scratch_types=[
                pltpu.VMEM((CBS,), jnp.int32),
                pltpu.VMEM((KBS, 128), jnp.int32),
                pltpu.VMEM((CBS, FEAT), jnp.float32),
                pltpu.VMEM_SHARED((SR, FEAT), jnp.float32),
                pltpu.SemaphoreType.DMA,
            ],
        )(_scatter_body)
    return _sc_cache["g"], _sc_cache["s"]


def _scatter_body(msg_hbm, dst_hbm, z_hbm, out_hbm, didx, idx2d, msgv, acc,
                  sem):
    c = lax.axis_index("c")
    t = lax.axis_index("s")
    lo = c * HALF
    pltpu.sync_copy(z_hbm.at[pl.ds(t * (SR // 16), SR // 16)],
                    acc.at[pl.ds(t * (SR // 16), SR // 16)])
    plsc.subcore_barrier()

    def outer(o, carry):
        base = t * (EP // 16) + o * CBS
        cp1 = pltpu.async_copy(dst_hbm.at[pl.ds(base, CBS)], didx, sem)
        cp2 = pltpu.async_copy(msg_hbm.at[pl.ds(base, CBS), :], msgv, sem)
        cp1.wait()
        cp2.wait()

        def conv_row(r, carry2):
            def conv16(q, carry3):
                v = didx[pl.ds((r * 8 + q) * 16, 16)]
                inr = (v >= lo) & (v < lo + HALF)
                idx2d[r, pl.ds(q * 16, 16)] = jnp.where(inr, v - lo,
                                                        HALF + t)
                return carry3

            return lax.fori_loop(0, 8, conv16, carry2)

        lax.fori_loop(0, KBS, conv_row, 0)
        adds = [
            pltpu.async_copy(msgv.at[pl.ds(j * 128, 128), :],
                             acc.at[idx2d.at[j]], sem, add=True)
            for j in range(KBS)
        ]
        for a in adds:
            a.wait()
        return carry

    lax.fori_loop(0, EP // 16 // CBS, outer, 0)
    plsc.subcore_barrier()
    pltpu.sync_copy(acc.at[pl.ds(t * (HALF // 16), HALF // 16)],
                    out_hbm.at[pl.ds(c * HALF + t * (HALF // 16),
                                     HALF // 16)])


def _sigmoid(x):
    return 1.0 / (1.0 + jnp.exp(-x))


def _k1_body(g_ref, w1, b1, w2, b2, out_ref):
    h = jnp.maximum(g_ref[...] @ w1[...] + b1[...], 0.0)
    out_ref[...] = h @ w2[...] + b2[...]


def _bd4(w):
    return jax.scipy.linalg.block_diag(w, w, w, w)


def _k2_body(xs_ref, xd_ref, w1, w2, b, rep, tile, out_ref):
    xs = xs_ref[...]
    mr = jnp.tanh(xs @ w1[...] + xd_ref[...] @ w2[...] + b[...])
    prod = (mr @ rep[...]) * (xs @ tile[...])
    acc = prod[:, :128]
    for j in range(1, 8):
        acc = acc + prod[:, 128 * j:128 * (j + 1)]
    out_ref[...] = acc


def _k3_body(x_ref, a_ref, w, b, out_ref):
    out_ref[...] = jnp.maximum((x_ref[...] - a_ref[...]) @ w[...] + b[...],
                               0.0)


def _k4_body(x_ref, w1, b1, w2s, b2, w3, b3, wcb1, bcb1, wcb2, bcb2, wt1,
             bt1, wt2, bt2, out_ref, ind_ref, comb_ref):
    x = x_ref[...]
    h1 = jnp.maximum(x @ w1[...] + b1[...], 0.0)
    h2s = [
        jnp.maximum(h1[:, 64 * p:64 * p + 64] @ w2s[p]
                    + b2[:, 64 * p:64 * p + 64], 0.0)
        for p in range(NPAT)
    ]
    h2 = jnp.concatenate(h2s, axis=1)
    ind = _sigmoid(h2 @ w3[...] + b3[...])
    ind_ref[...] = ind
    cb = jnp.maximum(ind @ wcb1[...] + bcb1[...], 0.0) @ wcb2[...] + bcb2[...]
    comb_ref[...] = _sigmoid(cb)
    feat = jnp.concatenate([x, ind], axis=1)
    out_ref[...] = (jnp.maximum(feat @ wt1[...] + bt1[...], 0.0) @ wt2[...]
                    + bt2[...])


def _full(shape):
    return pl.BlockSpec(shape, lambda i: tuple(0 for _ in shape))


def _k1(grid_p, Wf1, bf1, Wf2, bf2):
    return pl.pallas_call(
        _k1_body,
        grid=(N // NBL,),
        in_specs=[
            pl.BlockSpec((NBL // 4, 40), lambda i: (i, 0)),
            _full((40, 128)), _full((1, 128)),
            _full((128, 128)), _full((1, 128)),
        ],
        out_specs=pl.BlockSpec((NBL // 4, 128), lambda i: (i, 0)),
        out_shape=jax.ShapeDtypeStruct((N // 4, 128), jnp.float32),
    )(grid_p, _bd4(Wf1), jnp.tile(bf1, 4)[None], _bd4(Wf2),
      jnp.tile(bf2, 4)[None])


def _k2(xg_p, w1, w2, b):
    eb4 = EB // 4
    return pl.pallas_call(
        _k2_body,
        grid=(EP // EB,),
        in_specs=[
            pl.BlockSpec((eb4, 128), lambda i: (i, 0)),
            pl.BlockSpec((eb4, 128), lambda i: (i + EP // EB, 0)),
            _full((128, 256)), _full((128, 256)), _full((1, 256)),
            _full((256, 1024)), _full((128, 1024)),
        ],
        out_specs=pl.BlockSpec((eb4, 128), lambda i: (i, 0)),
        out_shape=jax.ShapeDtypeStruct((EP // 4, 128), jnp.float32),
    )(xg_p, xg_p, _bd4(w1), _bd4(w2), jnp.tile(b[0], 4)[None],
      jnp.asarray(_BREP), jnp.asarray(_BTILE))


def _k3(x_p, agg_p, w, b):
    return pl.pallas_call(
        _k3_body,
        grid=(N // NBL,),
        in_specs=[
            pl.BlockSpec((NBL // 4, 128), lambda i: (i, 0)),
            pl.BlockSpec((NBL // 4, 128), lambda i: (i, 0)),
            _full((128, 128)), _full((1, 128)),
        ],
        out_specs=pl.BlockSpec((NBL // 4, 128), lambda i: (i, 0)),
        out_shape=jax.ShapeDtypeStruct((N // 4, 128), jnp.float32),
    )(x_p, agg_p, _bd4(w), jnp.tile(b[0], 4)[None])


def _k4(x, w1, b1, w2s, b2, w3, b3, wcb1, bcb1, wcb2, bcb2, wt1, bt1, wt2,
        bt2):
    return pl.pallas_call(
        _k4_body,
        grid=(N // NB,),
        in_specs=[
            pl.BlockSpec((NB, FEAT), lambda i: (i, 0)),
            _full((FEAT, 512)), _full((1, 512)),
            _full((NPAT, 64, 64)), _full((1, 512)),
            _full((512, NPAT)), _full((1, NPAT)),
            _full((NPAT, 64)), _full((1, 64)),
            _full((64, 1)), _full((1, 1)),
            _full((FEAT + NPAT, FEAT)), _full((1, FEAT)),
            _full((FEAT, 10)), _full((1, 10)),
        ],
        out_specs=[
            pl.BlockSpec((NB, 10), lambda i: (i, 0)),
            pl.BlockSpec((NB, NPAT), lambda i: (i, 0)),
            pl.BlockSpec((NB, 1), lambda i: (i, 0)),
        ],
        out_shape=[
            jax.ShapeDtypeStruct((N, 10), jnp.float32),
            jax.ShapeDtypeStruct((N, NPAT), jnp.float32),
            jax.ShapeDtypeStruct((N, 1), jnp.float32),
        ],
    )(x, w1, b1, w2s, b2, w3, b3, wcb1, bcb1, wcb2, bcb2, wt1, bt1, wt2, bt2)


_q = np.arange(256)
_PERM = np.asarray(8 * (_q // 32) + (_q % 32) // 4, dtype=np.int32)
_REP = np.zeros((64, 256), np.float32)
_REP[_PERM, _q] = 1.0
_TILE = np.zeros((32, 256), np.float32)
_TILE[_q % 32, _q] = 1.0
_SRED = np.zeros((256, 32), np.float32)
_SRED[_q, 4 * (_q // 32) + _q % 4] = 1.0


_qq = np.arange(1024)
_j8 = _qq // 128
_r128 = _qq % 128
_u4 = _r128 // 32
_i8 = (_r128 % 32) // 4
_c4 = _qq % 4
_BREP = np.zeros((256, 1024), np.float32)
_BREP[64 * _u4 + 8 * _i8 + _j8, _qq] = 1.0
_BTILE = np.zeros((128, 1024), np.float32)
_BTILE[32 * _u4 + 4 * _j8 + _c4, _qq] = 1.0


def _np_bd4(w):
    k, m = w.shape
    out = np.zeros((4 * k, 4 * m), np.float32)
    for u in range(4):
        out[u * k:(u + 1) * k, u * m:(u + 1) * m] = w
    return out


def kernel(grid, edge_index, Wf1, bf1, Wf2, bf2, Wmap, bmap, Wl, bl, Wc1,
           bc1, Wc2, bc2, Wc3, bc3, Wcb1, bcb1, Wcb2, bcb2, Wt1, bt1, Wt2,
           bt2):
    pad = EP - E
    ei_g = jnp.concatenate(
        [edge_index, jnp.zeros((2, pad), jnp.int32)], axis=1)
    dst_s = jnp.concatenate(
        [edge_index[1], jnp.full((pad,), N, jnp.int32)])
    zrows = jnp.zeros((SR, FEAT), jnp.float32)

    gather_fn, scatter_fn = _get_sc_kernels()
    xp = _k1(grid.reshape(N // 4, 40), Wf1, bf1, Wf2, bf2)
    for l in range(2):
        xg = gather_fn(xp.reshape(N, FEAT), ei_g)
        msg_p = _k2(xg.reshape(2 * EP // 4, 128), Wmap[l][:FEAT],
                    Wmap[l][FEAT:], bmap[l][None])
        agg = scatter_fn(msg_p.reshape(EP, FEAT), dst_s, zrows)
        xp = _k3(xp, agg.reshape(N // 4, 128), Wl[l], bl[l][None])
    x = xp.reshape(N, FEAT)

    w1all = jnp.transpose(Wc1, (1, 0, 2)).reshape(FEAT, NPAT * 64)
    b1all = bc1.reshape(1, NPAT * 64)
    b2all = bc2.reshape(1, NPAT * 64)
    w3blk = jax.scipy.linalg.block_diag(*[Wc3[p] for p in range(NPAT)])
    b3all = bc3.reshape(1, NPAT)
    out, individual, combined = _k4(
        x, w1all, b1all, Wc2, b2all, w3blk, b3all, Wcb1, bcb1[None],
        Wcb2, bcb2[None], Wt1, bt1[None], Wt2, bt2[None])
    return out, individual, combined


# speedup vs baseline: 6.9863x; 1.1193x over previous
import functools

import jax
import jax.numpy as jnp
import numpy as np
from jax import lax
from jax.experimental import pallas as pl
from jax.experimental.pallas import tpu as pltpu
from jax.experimental.pallas import tpu_sc as plsc

N = 102400
E = 408320
EP = 409600
FEAT = 32
STALK = 8
CH = 4
NPAT = 8
HALF = N // 2
SR = HALF + 128
ROWS_W = 2 * EP // 32
CB = 2560
KB = CB // 128
CBS = 512
KBS = CBS // 128
EB = 2560
NB = 1024
NBL = 6400

def _gather_body(ne, eoff, x_hbm, ei_hbm, out_hbm, idx1, rows, sem):
    c = lax.axis_index("c")
    s = lax.axis_index("s")
    w = s * 2 + c
    h = w // 16
    t = w % 16
    rows_w = ne // 16

    def outer(o, carry):
        base = t * rows_w + o * CB
        pltpu.async_copy(ei_hbm.at[h, pl.ds(eoff + base, CB)], idx1,
                         sem).wait()
        gs = [
            pltpu.async_copy(x_hbm.at[idx1.at[pl.ds(j * 128, 128)]],
                             rows.at[pl.ds(j * 128, 128), :], sem)
            for j in range(KB)
        ]
        for g in gs:
            g.wait()
        pltpu.sync_copy(rows, out_hbm.at[pl.ds(h * ne + base, CB), :])
        return carry

    lax.fori_loop(0, rows_w // CB, outer, 0)


_sc_cache = {}


def _get_sc_kernels(ne, eoff):
    key = (ne, eoff)
    if key not in _sc_cache:
        mesh = plsc.VectorSubcoreMesh(core_axis_name="c",
                                      subcore_axis_name="s")
        params = pltpu.CompilerParams(use_tc_tiling_on_sc=False)
        g = functools.partial(
            pl.kernel,
            out_type=jax.ShapeDtypeStruct((2 * ne, FEAT), jnp.float32),
            mesh=mesh,
            compiler_params=params,
            scratch_types=[
                pltpu.VMEM((CB,), jnp.int32),
                pltpu.VMEM((CB, FEAT), jnp.float32),
                pltpu.SemaphoreType.DMA,
            ],
        )(functools.partial(_gather_body, ne, eoff))
        sc = functools.partial(
            pl.kernel,
            out_type=jax.ShapeDtypeStruct((N, FEAT), jnp.float32),
            mesh=mesh,
            compiler_params=params,
            scratch_types=[
                pltpu.VMEM((CBS,), jnp.int32),
                pltpu.VMEM((KBS, 128), jnp.int32),
                pltpu.VMEM((CBS, FEAT), jnp.float32),
                pltpu.VMEM_SHARED((SR, FEAT), jnp.float32),
                pltpu.SemaphoreType.DMA,
            ],
        )(functools.partial(_scatter_body, ne, eoff))
        _sc_cache[key] = (g, sc)
    return _sc_cache[key]


def _scatter_body(ne, eoff, msg_hbm, dst_hbm, z_hbm, out_hbm, didx, idx2d,
                  msgv, acc, sem):
    c = lax.axis_index("c")
    t = lax.axis_index("s")
    lo = c * HALF
    pltpu.sync_copy(z_hbm.at[pl.ds(t * (SR // 16), SR // 16)],
                    acc.at[pl.ds(t * (SR // 16), SR // 16)])
    plsc.subcore_barrier()

    def outer(o, carry):
        base = t * (ne // 16) + o * CBS
        cp1 = pltpu.async_copy(dst_hbm.at[pl.ds(eoff + base, CBS)], didx,
                               sem)
        cp2 = pltpu.async_copy(msg_hbm.at[pl.ds(base, CBS), :], msgv, sem)
        cp1.wait()
        cp2.wait()

        def conv_row(r, carry2):
            def conv16(q, carry3):
                v = didx[pl.ds((r * 8 + q) * 16, 16)]
                inr = (v >= lo) & (v < lo + HALF)
                idx2d[r, pl.ds(q * 16, 16)] = jnp.where(inr, v - lo,
                                                        HALF + t)
                return carry3

            return lax.fori_loop(0, 8, conv16, carry2)

        lax.fori_loop(0, KBS, conv_row, 0)
        adds = [
            pltpu.async_copy(msgv.at[pl.ds(j * 128, 128), :],
                             acc.at[idx2d.at[j]], sem, add=True)
            for j in range(KBS)
        ]
        for a in adds:
            a.wait()
        return carry

    lax.fori_loop(0, ne // 16 // CBS, outer, 0)
    plsc.subcore_barrier()
    pltpu.sync_copy(acc.at[pl.ds(t * (HALF // 16), HALF // 16)],
                    out_hbm.at[pl.ds(c * HALF + t * (HALF // 16),
                                     HALF // 16)])


def _sigmoid(x):
    return 1.0 / (1.0 + jnp.exp(-x))


def _k1_body(g_ref, w1, b1, w2, b2, out_ref):
    h = jnp.maximum(g_ref[...] @ w1[...] + b1[...], 0.0)
    out_ref[...] = h @ w2[...] + b2[...]


def _bd4(w):
    return jax.scipy.linalg.block_diag(w, w, w, w)


def _k2_body(xs_ref, xd_ref, w1, w2, b, rep, tile, out_ref):
    xs = xs_ref[...]
    mr = jnp.tanh(xs @ w1[...] + xd_ref[...] @ w2[...] + b[...])
    prod = (mr @ rep[...]) * (xs @ tile[...])
    acc = prod[:, :128]
    for j in range(1, 8):
        acc = acc + prod[:, 128 * j:128 * (j + 1)]
    out_ref[...] = acc


def _k3_body(x_ref, a_ref, a2_ref, w, b, out_ref):
    out_ref[...] = jnp.maximum(
        (x_ref[...] - a_ref[...] - a2_ref[...]) @ w[...] + b[...], 0.0)


def _k4_body(x_ref, w1, b1, w2s, b2, w3, b3, wcb1, bcb1, wcb2, bcb2, wt1,
             bt1, wt2, bt2, out_ref, ind_ref, comb_ref):
    x = x_ref[...]
    h1 = jnp.maximum(x @ w1[...] + b1[...], 0.0)
    h2s = [
        jnp.maximum(h1[:, 64 * p:64 * p + 64] @ w2s[p]
                    + b2[:, 64 * p:64 * p + 64], 0.0)
        for p in range(NPAT)
    ]
    h2 = jnp.concatenate(h2s, axis=1)
    ind = _sigmoid(h2 @ w3[...] + b3[...])
    ind_ref[...] = ind
    cb = jnp.maximum(ind @ wcb1[...] + bcb1[...], 0.0) @ wcb2[...] + bcb2[...]
    comb_ref[...] = _sigmoid(cb)
    feat = jnp.concatenate([x, ind], axis=1)
    out_ref[...] = (jnp.maximum(feat @ wt1[...] + bt1[...], 0.0) @ wt2[...]
                    + bt2[...])


def _full(shape):
    return pl.BlockSpec(shape, lambda i: tuple(0 for _ in shape))


def _k1(grid_p, Wf1, bf1, Wf2, bf2):
    return pl.pallas_call(
        _k1_body,
        grid=(N // NBL,),
        in_specs=[
            pl.BlockSpec((NBL // 4, 40), lambda i: (i, 0)),
            _full((40, 128)), _full((1, 128)),
            _full((128, 128)), _full((1, 128)),
        ],
        out_specs=pl.BlockSpec((NBL // 4, 128), lambda i: (i, 0)),
        out_shape=jax.ShapeDtypeStruct((N // 4, 128), jnp.float32),
    )(grid_p, _bd4(Wf1), jnp.tile(bf1, 4)[None], _bd4(Wf2),
      jnp.tile(bf2, 4)[None])


def _k2(xg_p, w1, w2, b, ne):
    eb4 = EB // 4
    nblk = ne // EB
    return pl.pallas_call(
        _k2_body,
        grid=(ne // EB,),
        in_specs=[
            pl.BlockSpec((eb4, 128), lambda i: (i, 0)),
            pl.BlockSpec((eb4, 128), lambda i: (i + nblk, 0)),
            _full((128, 256)), _full((128, 256)), _full((1, 256)),
            _full((256, 1024)), _full((128, 1024)),
        ],
        out_specs=pl.BlockSpec((eb4, 128), lambda i: (i, 0)),
        out_shape=jax.ShapeDtypeStruct((ne // 4, 128), jnp.float32),
    )(xg_p, xg_p, _bd4(w1), _bd4(w2), jnp.tile(b[0], 4)[None],
      jnp.asarray(_BREP), jnp.asarray(_BTILE))


def _k3(x_p, agg_p, agg2_p, w, b):
    return pl.pallas_call(
        _k3_body,
        grid=(N // NBL,),
        in_specs=[
            pl.BlockSpec((NBL // 4, 128), lambda i: (i, 0)),
            pl.BlockSpec((NBL // 4, 128), lambda i: (i, 0)),
            pl.BlockSpec((NBL // 4, 128), lambda i: (i, 0)),
            _full((128, 128)), _full((1, 128)),
        ],
        out_specs=pl.BlockSpec((NBL // 4, 128), lambda i: (i, 0)),
        out_shape=jax.ShapeDtypeStruct((N // 4, 128), jnp.float32),
    )(x_p, agg_p, agg2_p, _bd4(w), jnp.tile(b[0], 4)[None])


def _k4(x, w1, b1, w2s, b2, w3, b3, wcb1, bcb1, wcb2, bcb2, wt1, bt1, wt2,
        bt2):
    return pl.pallas_call(
        _k4_body,
        grid=(N // NB,),
        in_specs=[
            pl.BlockSpec((NB, FEAT), lambda i: (i, 0)),
            _full((FEAT, 512)), _full((1, 512)),
            _full((NPAT, 64, 64)), _full((1, 512)),
            _full((512, NPAT)), _full((1, NPAT)),
            _full((NPAT, 64)), _full((1, 64)),
            _full((64, 1)), _full((1, 1)),
            _full((FEAT + NPAT, FEAT)), _full((1, FEAT)),
            _full((FEAT, 10)), _full((1, 10)),
        ],
        out_specs=[
            pl.BlockSpec((NB, 10), lambda i: (i, 0)),
            pl.BlockSpec((NB, NPAT), lambda i: (i, 0)),
            pl.BlockSpec((NB, 1), lambda i: (i, 0)),
        ],
        out_shape=[
            jax.ShapeDtypeStruct((N, 10), jnp.float32),
            jax.ShapeDtypeStruct((N, NPAT), jnp.float32),
            jax.ShapeDtypeStruct((N, 1), jnp.float32),
        ],
    )(x, w1, b1, w2s, b2, w3, b3, wcb1, bcb1, wcb2, bcb2, wt1, bt1, wt2, bt2)


_q = np.arange(256)
_PERM = np.asarray(8 * (_q // 32) + (_q % 32) // 4, dtype=np.int32)
_REP = np.zeros((64, 256), np.float32)
_REP[_PERM, _q] = 1.0
_TILE = np.zeros((32, 256), np.float32)
_TILE[_q % 32, _q] = 1.0
_SRED = np.zeros((256, 32), np.float32)
_SRED[_q, 4 * (_q // 32) + _q % 4] = 1.0


_qq = np.arange(1024)
_j8 = _qq // 128
_r128 = _qq % 128
_u4 = _r128 // 32
_i8 = (_r128 % 32) // 4
_c4 = _qq % 4
_BREP = np.zeros((256, 1024), np.float32)
_BREP[64 * _u4 + 8 * _i8 + _j8, _qq] = 1.0
_BTILE = np.zeros((128, 1024), np.float32)
_BTILE[32 * _u4 + 4 * _j8 + _c4, _qq] = 1.0


def _np_bd4(w):
    k, m = w.shape
    out = np.zeros((4 * k, 4 * m), np.float32)
    for u in range(4):
        out[u * k:(u + 1) * k, u * m:(u + 1) * m] = w
    return out


def kernel(grid, edge_index, Wf1, bf1, Wf2, bf2, Wmap, bmap, Wl, bl, Wc1,
           bc1, Wc2, bc2, Wc3, bc3, Wcb1, bcb1, Wcb2, bcb2, Wt1, bt1, Wt2,
           bt2):
    pad = EP - E
    ei_g = jnp.concatenate(
        [edge_index, jnp.zeros((2, pad), jnp.int32)], axis=1)
    dst_s = jnp.concatenate(
        [edge_index[1], jnp.full((pad,), N, jnp.int32)])
    zrows = jnp.zeros((SR, FEAT), jnp.float32)

    NE = EP // 2
    g_a, s_a = _get_sc_kernels(NE, 0)
    g_b, s_b = _get_sc_kernels(NE, NE)
    xp = _k1(grid.reshape(N // 4, 40), Wf1, bf1, Wf2, bf2)
    for l in range(2):
        x_lin = xp.reshape(N, FEAT)
        w1l, w2l, bml = Wmap[l][:FEAT], Wmap[l][FEAT:], bmap[l][None]
        xga = g_a(x_lin, ei_g)
        xgb = g_b(x_lin, ei_g)
        msg_a = _k2(xga.reshape(2 * NE // 4, 128), w1l, w2l, bml, NE)
        msg_b = _k2(xgb.reshape(2 * NE // 4, 128), w1l, w2l, bml, NE)
        agg_a = s_a(msg_a.reshape(NE, FEAT), dst_s, zrows)
        agg_b = s_b(msg_b.reshape(NE, FEAT), dst_s, zrows)
        xp = _k3(xp, agg_a.reshape(N // 4, 128), agg_b.reshape(N // 4, 128),
                 Wl[l], bl[l][None])
    x = xp.reshape(N, FEAT)

    w1all = jnp.transpose(Wc1, (1, 0, 2)).reshape(FEAT, NPAT * 64)
    b1all = bc1.reshape(1, NPAT * 64)
    b2all = bc2.reshape(1, NPAT * 64)
    w3blk = jax.scipy.linalg.block_diag(*[Wc3[p] for p in range(NPAT)])
    b3all = bc3.reshape(1, NPAT)
    out, individual, combined = _k4(
        x, w1all, b1all, Wc2, b2all, w3blk, b3all, Wcb1, bcb1[None],
        Wcb2, bcb2[None], Wt1, bt1[None], Wt2, bt2[None])
    return out, individual, combined


# speedup vs baseline: 7.0069x; 1.0030x over previous
import functools

import jax
import jax.numpy as jnp
import numpy as np
from jax import lax
from jax.experimental import pallas as pl
from jax.experimental.pallas import tpu as pltpu
from jax.experimental.pallas import tpu_sc as plsc

N = 102400
E = 408320
EP = 409600
FEAT = 32
STALK = 8
CH = 4
NPAT = 8
HALF = N // 2
SR = HALF + 128
ROWS_W = 2 * EP // 32
CB = 2560
KB = CB // 128
CBS = 640
KBS = CBS // 128
EB = 2560
NB = 1024
NBL = 6400

def _gather_body(ne, eoff, x_hbm, ei_hbm, out_hbm, idx1, rows, sem):
    c = lax.axis_index("c")
    s = lax.axis_index("s")
    w = s * 2 + c
    h = w // 16
    t = w % 16
    rows_w = ne // 16

    def outer(o, carry):
        base = t * rows_w + o * CB
        pltpu.async_copy(ei_hbm.at[h, pl.ds(eoff + base, CB)], idx1,
                         sem).wait()
        gs = [
            pltpu.async_copy(x_hbm.at[idx1.at[pl.ds(j * 128, 128)]],
                             rows.at[pl.ds(j * 128, 128), :], sem)
            for j in range(KB)
        ]
        for g in gs:
            g.wait()
        pltpu.sync_copy(rows, out_hbm.at[pl.ds(h * ne + base, CB), :])
        return carry

    lax.fori_loop(0, rows_w // CB, outer, 0)


_sc_cache = {}


def _get_sc_kernels(ne, eoff):
    key = (ne, eoff)
    if key not in _sc_cache:
        mesh = plsc.VectorSubcoreMesh(core_axis_name="c",
                                      subcore_axis_name="s")
        params = pltpu.CompilerParams(use_tc_tiling_on_sc=False)
        g = functools.partial(
            pl.kernel,
            out_type=jax.ShapeDtypeStruct((2 * ne, FEAT), jnp.float32),
            mesh=mesh,
            compiler_params=params,
            scratch_types=[
                pltpu.VMEM((CB,), jnp.int32),
                pltpu.VMEM((CB, FEAT), jnp.float32),
                pltpu.SemaphoreType.DMA,
            ],
        )(functools.partial(_gather_body, ne, eoff))
        sc = functools.partial(
            pl.kernel,
            out_type=jax.ShapeDtypeStruct((N, FEAT), jnp.float32),
            mesh=mesh,
            compiler_params=params,
            scratch_types=[
                pltpu.VMEM((CBS,), jnp.int32),
                pltpu.VMEM((KBS, 128), jnp.int32),
                pltpu.VMEM((CBS, FEAT), jnp.float32),
                pltpu.VMEM_SHARED((SR, FEAT), jnp.float32),
                pltpu.SemaphoreType.DMA,
            ],
        )(functools.partial(_scatter_body, ne, eoff))
        _sc_cache[key] = (g, sc)
    return _sc_cache[key]


def _scatter_body(ne, eoff, msg_hbm, dst_hbm, z_hbm, out_hbm, didx, idx2d,
                  msgv, acc, sem):
    c = lax.axis_index("c")
    t = lax.axis_index("s")
    lo = c * HALF
    pltpu.sync_copy(z_hbm.at[pl.ds(t * (SR // 16), SR // 16)],
                    acc.at[pl.ds(t * (SR // 16), SR // 16)])
    plsc.subcore_barrier()

    def outer(o, carry):
        base = t * (ne // 16) + o * CBS
        cp1 = pltpu.async_copy(dst_hbm.at[pl.ds(eoff + base, CBS)], didx,
                               sem)
        cp2 = pltpu.async_copy(msg_hbm.at[pl.ds(base, CBS), :], msgv, sem)
        cp1.wait()
        cp2.wait()

        def conv_row(r, carry2):
            def conv16(q, carry3):
                v = didx[pl.ds((r * 8 + q) * 16, 16)]
                inr = (v >= lo) & (v < lo + HALF)
                idx2d[r, pl.ds(q * 16, 16)] = jnp.where(inr, v - lo,
                                                        HALF + t)
                return carry3

            return lax.fori_loop(0, 8, conv16, carry2)

        lax.fori_loop(0, KBS, conv_row, 0)
        adds = [
            pltpu.async_copy(msgv.at[pl.ds(j * 128, 128), :],
                             acc.at[idx2d.at[j]], sem, add=True)
            for j in range(KBS)
        ]
        for a in adds:
            a.wait()
        return carry

    lax.fori_loop(0, ne // 16 // CBS, outer, 0)
    plsc.subcore_barrier()
    pltpu.sync_copy(acc.at[pl.ds(t * (HALF // 16), HALF // 16)],
                    out_hbm.at[pl.ds(c * HALF + t * (HALF // 16),
                                     HALF // 16)])


def _sigmoid(x):
    return 1.0 / (1.0 + jnp.exp(-x))


def _k1_body(g_ref, w1, b1, w2, b2, out_ref):
    h = jnp.maximum(g_ref[...] @ w1[...] + b1[...], 0.0)
    out_ref[...] = h @ w2[...] + b2[...]


def _bd4(w):
    return jax.scipy.linalg.block_diag(w, w, w, w)


def _k2_body(xs_ref, xd_ref, w1, w2, b, rep, tile, out_ref):
    xs = xs_ref[...]
    mr = jnp.tanh(xs @ w1[...] + xd_ref[...] @ w2[...] + b[...])
    mrr = jax.lax.dot(mr.astype(jnp.bfloat16), rep[...],
                      preferred_element_type=jnp.float32)
    xsr = jax.lax.dot(xs.astype(jnp.bfloat16), tile[...],
                      preferred_element_type=jnp.float32)
    prod = mrr * xsr
    acc = prod[:, :128]
    for j in range(1, 8):
        acc = acc + prod[:, 128 * j:128 * (j + 1)]
    out_ref[...] = acc


def _k3_body(x_ref, a_ref, a2_ref, w, b, out_ref):
    out_ref[...] = jnp.maximum(
        (x_ref[...] - a_ref[...] - a2_ref[...]) @ w[...] + b[...], 0.0)


def _k4_body(x_ref, w1, b1, w2s, b2, w3, b3, wcb1, bcb1, wcb2, bcb2, wt1,
             bt1, wt2, bt2, out_ref, ind_ref, comb_ref):
    x = x_ref[...]
    h1 = jnp.maximum(x @ w1[...] + b1[...], 0.0)
    h2s = [
        jnp.maximum(h1[:, 64 * p:64 * p + 64] @ w2s[p]
                    + b2[:, 64 * p:64 * p + 64], 0.0)
        for p in range(NPAT)
    ]
    h2 = jnp.concatenate(h2s, axis=1)
    ind = _sigmoid(h2 @ w3[...] + b3[...])
    ind_ref[...] = ind
    cb = jnp.maximum(ind @ wcb1[...] + bcb1[...], 0.0) @ wcb2[...] + bcb2[...]
    comb_ref[...] = _sigmoid(cb)
    feat = jnp.concatenate([x, ind], axis=1)
    out_ref[...] = (jnp.maximum(feat @ wt1[...] + bt1[...], 0.0) @ wt2[...]
                    + bt2[...])


def _full(shape, dtype=None):
    del dtype
    return pl.BlockSpec(shape, lambda i: tuple(0 for _ in shape))


def _k1(grid_p, Wf1, bf1, Wf2, bf2):
    return pl.pallas_call(
        _k1_body,
        grid=(N // NBL,),
        in_specs=[
            pl.BlockSpec((NBL // 4, 40), lambda i: (i, 0)),
            _full((40, 128)), _full((1, 128)),
            _full((128, 128)), _full((1, 128)),
        ],
        out_specs=pl.BlockSpec((NBL // 4, 128), lambda i: (i, 0)),
        out_shape=jax.ShapeDtypeStruct((N // 4, 128), jnp.float32),
    )(grid_p, _bd4(Wf1), jnp.tile(bf1, 4)[None], _bd4(Wf2),
      jnp.tile(bf2, 4)[None])


def _k2(xg_p, w1, w2, b, ne):
    eb4 = EB // 4
    nblk = ne // EB
    return pl.pallas_call(
        _k2_body,
        grid=(ne // EB,),
        in_specs=[
            pl.BlockSpec((eb4, 128), lambda i: (i, 0)),
            pl.BlockSpec((eb4, 128), lambda i: (i + nblk, 0)),
            _full((128, 256)), _full((128, 256)), _full((1, 256)),
            _full((256, 1024), jnp.bfloat16), _full((128, 1024),
                                                    jnp.bfloat16),
        ],
        out_specs=pl.BlockSpec((eb4, 128), lambda i: (i, 0)),
        out_shape=jax.ShapeDtypeStruct((ne // 4, 128), jnp.float32),
    )(xg_p, xg_p, _bd4(w1), _bd4(w2), jnp.tile(b[0], 4)[None],
      jnp.asarray(_BREP, jnp.bfloat16), jnp.asarray(_BTILE, jnp.bfloat16))


def _k3(x_p, agg_p, agg2_p, w, b):
    return pl.pallas_call(
        _k3_body,
        grid=(N // NBL,),
        in_specs=[
            pl.BlockSpec((NBL // 4, 128), lambda i: (i, 0)),
            pl.BlockSpec((NBL // 4, 128), lambda i: (i, 0)),
            pl.BlockSpec((NBL // 4, 128), lambda i: (i, 0)),
            _full((128, 128)), _full((1, 128)),
        ],
        out_specs=pl.BlockSpec((NBL // 4, 128), lambda i: (i, 0)),
        out_shape=jax.ShapeDtypeStruct((N // 4, 128), jnp.float32),
    )(x_p, agg_p, agg2_p, _bd4(w), jnp.tile(b[0], 4)[None])


def _k4(x, w1, b1, w2s, b2, w3, b3, wcb1, bcb1, wcb2, bcb2, wt1, bt1, wt2,
        bt2):
    return pl.pallas_call(
        _k4_body,
        grid=(N // NB,),
        in_specs=[
            pl.BlockSpec((NB, FEAT), lambda i: (i, 0)),
            _full((FEAT, 512)), _full((1, 512)),
            _full((NPAT, 64, 64)), _full((1, 512)),
            _full((512, NPAT)), _full((1, NPAT)),
            _full((NPAT, 64)), _full((1, 64)),
            _full((64, 1)), _full((1, 1)),
            _full((FEAT + NPAT, FEAT)), _full((1, FEAT)),
            _full((FEAT, 10)), _full((1, 10)),
        ],
        out_specs=[
            pl.BlockSpec((NB, 10), lambda i: (i, 0)),
            pl.BlockSpec((NB, NPAT), lambda i: (i, 0)),
            pl.BlockSpec((NB, 1), lambda i: (i, 0)),
        ],
        out_shape=[
            jax.ShapeDtypeStruct((N, 10), jnp.float32),
            jax.ShapeDtypeStruct((N, NPAT), jnp.float32),
            jax.ShapeDtypeStruct((N, 1), jnp.float32),
        ],
    )(x, w1, b1, w2s, b2, w3, b3, wcb1, bcb1, wcb2, bcb2, wt1, bt1, wt2, bt2)


_q = np.arange(256)
_PERM = np.asarray(8 * (_q // 32) + (_q % 32) // 4, dtype=np.int32)
_REP = np.zeros((64, 256), np.float32)
_REP[_PERM, _q] = 1.0
_TILE = np.zeros((32, 256), np.float32)
_TILE[_q % 32, _q] = 1.0
_SRED = np.zeros((256, 32), np.float32)
_SRED[_q, 4 * (_q // 32) + _q % 4] = 1.0


_qq = np.arange(1024)
_j8 = _qq // 128
_r128 = _qq % 128
_u4 = _r128 // 32
_i8 = (_r128 % 32) // 4
_c4 = _qq % 4
_BREP = np.zeros((256, 1024), np.float32)
_BREP[64 * _u4 + 8 * _i8 + _j8, _qq] = 1.0
_BTILE = np.zeros((128, 1024), np.float32)
_BTILE[32 * _u4 + 4 * _j8 + _c4, _qq] = 1.0


def _np_bd4(w):
    k, m = w.shape
    out = np.zeros((4 * k, 4 * m), np.float32)
    for u in range(4):
        out[u * k:(u + 1) * k, u * m:(u + 1) * m] = w
    return out


def kernel(grid, edge_index, Wf1, bf1, Wf2, bf2, Wmap, bmap, Wl, bl, Wc1,
           bc1, Wc2, bc2, Wc3, bc3, Wcb1, bcb1, Wcb2, bcb2, Wt1, bt1, Wt2,
           bt2):
    pad = EP - E
    ei_g = jnp.concatenate(
        [edge_index, jnp.zeros((2, pad), jnp.int32)], axis=1)
    dst_s = jnp.concatenate(
        [edge_index[1], jnp.full((pad,), N, jnp.int32)])
    zrows = jnp.zeros((SR, FEAT), jnp.float32)

    NE = EP // 2
    g_a, s_a = _get_sc_kernels(NE, 0)
    g_b, s_b = _get_sc_kernels(NE, NE)
    xp = _k1(grid.reshape(N // 4, 40), Wf1, bf1, Wf2, bf2)
    for l in range(2):
        x_lin = xp.reshape(N, FEAT)
        w1l, w2l, bml = Wmap[l][:FEAT], Wmap[l][FEAT:], bmap[l][None]
        xga = g_a(x_lin, ei_g)
        xgb = g_b(x_lin, ei_g)
        msg_a = _k2(xga.reshape(2 * NE // 4, 128), w1l, w2l, bml, NE)
        msg_b = _k2(xgb.reshape(2 * NE // 4, 128), w1l, w2l, bml, NE)
        agg_a = s_a(msg_a.reshape(NE, FEAT), dst_s, zrows)
        agg_b = s_b(msg_b.reshape(NE, FEAT), dst_s, zrows)
        xp = _k3(xp, agg_a.reshape(N // 4, 128), agg_b.reshape(N // 4, 128),
                 Wl[l], bl[l][None])
    x = xp.reshape(N, FEAT)

    w1all = jnp.transpose(Wc1, (1, 0, 2)).reshape(FEAT, NPAT * 64)
    b1all = bc1.reshape(1, NPAT * 64)
    b2all = bc2.reshape(1, NPAT * 64)
    w3blk = jax.scipy.linalg.block_diag(*[Wc3[p] for p in range(NPAT)])
    b3all = bc3.reshape(1, NPAT)
    out, individual, combined = _k4(
        x, w1all, b1all, Wc2, b2all, w3blk, b3all, Wcb1, bcb1[None],
        Wcb2, bcb2[None], Wt1, bt1[None], Wt2, bt2[None])
    return out, individual, combined


# speedup vs baseline: 7.2947x; 1.0411x over previous
import functools

import jax
import jax.numpy as jnp
import numpy as np
from jax import lax
from jax.experimental import pallas as pl
from jax.experimental.pallas import tpu as pltpu
from jax.experimental.pallas import tpu_sc as plsc

N = 102400
E = 408320
EP = 409600
FEAT = 32
STALK = 8
CH = 4
NPAT = 8
HALF = N // 2
SR = HALF + 128
ROWS_W = 2 * EP // 32
CB = 2560
KB = CB // 128
CBS = 640
KBS = CBS // 128
EB = 2560
NB = 2048
NBL = 6400

def _gather_body(ne, eoff, x_hbm, ei_hbm, out_hbm, idx1, rows, sem):
    c = lax.axis_index("c")
    s = lax.axis_index("s")
    w = s * 2 + c
    h = w // 16
    t = w % 16
    rows_w = ne // 16

    def outer(o, carry):
        base = t * rows_w + o * CB
        pltpu.async_copy(ei_hbm.at[h, pl.ds(eoff + base, CB)], idx1,
                         sem).wait()
        gs = [
            pltpu.async_copy(x_hbm.at[idx1.at[pl.ds(j * 128, 128)]],
                             rows.at[pl.ds(j * 128, 128), :], sem)
            for j in range(KB)
        ]
        for g in gs:
            g.wait()
        pltpu.sync_copy(rows, out_hbm.at[pl.ds(h * ne + base, CB), :])
        return carry

    lax.fori_loop(0, rows_w // CB, outer, 0)


_sc_cache = {}


def _get_sc_kernels(ne, eoff):
    key = (ne, eoff)
    if key not in _sc_cache:
        mesh = plsc.VectorSubcoreMesh(core_axis_name="c",
                                      subcore_axis_name="s")
        params = pltpu.CompilerParams(use_tc_tiling_on_sc=False)
        g = functools.partial(
            pl.kernel,
            out_type=jax.ShapeDtypeStruct((2 * ne, FEAT), jnp.float32),
            mesh=mesh,
            compiler_params=params,
            scratch_types=[
                pltpu.VMEM((CB,), jnp.int32),
                pltpu.VMEM((CB, FEAT), jnp.float32),
                pltpu.SemaphoreType.DMA,
            ],
        )(functools.partial(_gather_body, ne, eoff))
        sc = functools.partial(
            pl.kernel,
            out_type=jax.ShapeDtypeStruct((N, FEAT), jnp.float32),
            mesh=mesh,
            compiler_params=params,
            scratch_types=[
                pltpu.VMEM((CBS,), jnp.int32),
                pltpu.VMEM((KBS, 128), jnp.int32),
                pltpu.VMEM((CBS, FEAT), jnp.float32),
                pltpu.VMEM_SHARED((SR, FEAT), jnp.float32),
                pltpu.SemaphoreType.DMA,
            ],
        )(functools.partial(_scatter_body, ne, eoff))
        _sc_cache[key] = (g, sc)
    return _sc_cache[key]


def _scatter_body(ne, eoff, msg_hbm, dst_hbm, z_hbm, out_hbm, didx, idx2d,
                  msgv, acc, sem):
    c = lax.axis_index("c")
    t = lax.axis_index("s")
    lo = c * HALF
    pltpu.sync_copy(z_hbm.at[pl.ds(t * (SR // 16), SR // 16)],
                    acc.at[pl.ds(t * (SR // 16), SR // 16)])
    plsc.subcore_barrier()

    def outer(o, carry):
        base = t * (ne // 16) + o * CBS
        cp1 = pltpu.async_copy(dst_hbm.at[pl.ds(eoff + base, CBS)], didx,
                               sem)
        cp2 = pltpu.async_copy(msg_hbm.at[pl.ds(base, CBS), :], msgv, sem)
        cp1.wait()
        cp2.wait()

        def conv_row(r, carry2):
            def conv16(q, carry3):
                v = didx[pl.ds((r * 8 + q) * 16, 16)]
                inr = (v >= lo) & (v < lo + HALF)
                idx2d[r, pl.ds(q * 16, 16)] = jnp.where(inr, v - lo,
                                                        HALF + t)
                return carry3

            return lax.fori_loop(0, 8, conv16, carry2)

        lax.fori_loop(0, KBS, conv_row, 0)
        adds = [
            pltpu.async_copy(msgv.at[pl.ds(j * 128, 128), :],
                             acc.at[idx2d.at[j]], sem, add=True)
            for j in range(KBS)
        ]
        for a in adds:
            a.wait()
        return carry

    lax.fori_loop(0, ne // 16 // CBS, outer, 0)
    plsc.subcore_barrier()
    pltpu.sync_copy(acc.at[pl.ds(t * (HALF // 16), HALF // 16)],
                    out_hbm.at[pl.ds(c * HALF + t * (HALF // 16),
                                     HALF // 16)])


def _sigmoid(x):
    return 1.0 / (1.0 + jnp.exp(-x))


def _k1_body(g_ref, w1, b1, w2, b2, out_ref):
    h = jnp.maximum(g_ref[...] @ w1[...] + b1[...], 0.0)
    out_ref[...] = h @ w2[...] + b2[...]


def _bd4(w):
    return jax.scipy.linalg.block_diag(w, w, w, w)


def _k2_body(xs_ref, xd_ref, w1, w2, b, rep, tile, out_ref):
    xs = xs_ref[...]
    mr = jnp.tanh(xs @ w1[...] + xd_ref[...] @ w2[...] + b[...])
    mrr = jax.lax.dot(mr.astype(jnp.bfloat16), rep[...],
                      preferred_element_type=jnp.float32)
    xsr = jax.lax.dot(xs.astype(jnp.bfloat16), tile[...],
                      preferred_element_type=jnp.float32)
    prod = mrr * xsr
    acc = prod[:, :128]
    for j in range(1, 8):
        acc = acc + prod[:, 128 * j:128 * (j + 1)]
    out_ref[...] = acc


def _k3_body(x_ref, a_ref, a2_ref, w, b, out_ref):
    out_ref[...] = jnp.maximum(
        (x_ref[...] - a_ref[...] - a2_ref[...]) @ w[...] + b[...], 0.0)


def _k4_body(x_ref, w1, b1, w2s, b2, w3, b3, wcb1, bcb1, wcb2, bcb2, wt1,
             bt1, wt2, bt2, out_ref, ind_ref, comb_ref):
    x = x_ref[...]
    h1 = jnp.maximum(x @ w1[...] + b1[...], 0.0)
    h2s = [
        jnp.maximum(h1[:, 128 * p:128 * p + 128] @ w2s[p]
                    + b2[:, 128 * p:128 * p + 128], 0.0)
        for p in range(NPAT // 2)
    ]
    h2 = jnp.concatenate(h2s, axis=1)
    ind = _sigmoid(h2 @ w3[...] + b3[...])
    ind_ref[...] = ind
    cb = jnp.maximum(ind @ wcb1[...] + bcb1[...], 0.0) @ wcb2[...] + bcb2[...]
    comb_ref[...] = _sigmoid(cb)
    feat = jnp.concatenate([x, ind], axis=1)
    out_ref[...] = (jnp.maximum(feat @ wt1[...] + bt1[...], 0.0) @ wt2[...]
                    + bt2[...])


def _full(shape, dtype=None):
    del dtype
    return pl.BlockSpec(shape, lambda i: tuple(0 for _ in shape))


def _k1(grid_p, Wf1, bf1, Wf2, bf2):
    return pl.pallas_call(
        _k1_body,
        grid=(N // NBL,),
        in_specs=[
            pl.BlockSpec((NBL // 4, 40), lambda i: (i, 0)),
            _full((40, 128)), _full((1, 128)),
            _full((128, 128)), _full((1, 128)),
        ],
        out_specs=pl.BlockSpec((NBL // 4, 128), lambda i: (i, 0)),
        out_shape=jax.ShapeDtypeStruct((N // 4, 128), jnp.float32),
    )(grid_p, _bd4(Wf1), jnp.tile(bf1, 4)[None], _bd4(Wf2),
      jnp.tile(bf2, 4)[None])


def _k2(xg_p, w1, w2, b, ne):
    eb4 = EB // 4
    nblk = ne // EB
    return pl.pallas_call(
        _k2_body,
        grid=(ne // EB,),
        in_specs=[
            pl.BlockSpec((eb4, 128), lambda i: (i, 0)),
            pl.BlockSpec((eb4, 128), lambda i: (i + nblk, 0)),
            _full((128, 256)), _full((128, 256)), _full((1, 256)),
            _full((256, 1024), jnp.bfloat16), _full((128, 1024),
                                                    jnp.bfloat16),
        ],
        out_specs=pl.BlockSpec((eb4, 128), lambda i: (i, 0)),
        out_shape=jax.ShapeDtypeStruct((ne // 4, 128), jnp.float32),
    )(xg_p, xg_p, _bd4(w1), _bd4(w2), jnp.tile(b[0], 4)[None],
      jnp.asarray(_BREP, jnp.bfloat16), jnp.asarray(_BTILE, jnp.bfloat16))


def _k3(x_p, agg_p, agg2_p, w, b):
    return pl.pallas_call(
        _k3_body,
        grid=(N // NBL,),
        in_specs=[
            pl.BlockSpec((NBL // 4, 128), lambda i: (i, 0)),
            pl.BlockSpec((NBL // 4, 128), lambda i: (i, 0)),
            pl.BlockSpec((NBL // 4, 128), lambda i: (i, 0)),
            _full((128, 128)), _full((1, 128)),
        ],
        out_specs=pl.BlockSpec((NBL // 4, 128), lambda i: (i, 0)),
        out_shape=jax.ShapeDtypeStruct((N // 4, 128), jnp.float32),
    )(x_p, agg_p, agg2_p, _bd4(w), jnp.tile(b[0], 4)[None])


def _k4(x, w1, b1, w2s, b2, w3, b3, wcb1, bcb1, wcb2, bcb2, wt1, bt1, wt2,
        bt2):
    return pl.pallas_call(
        _k4_body,
        grid=(N // NB,),
        in_specs=[
            pl.BlockSpec((NB, FEAT), lambda i: (i, 0)),
            _full((FEAT, 512)), _full((1, 512)),
            _full((NPAT // 2, 128, 128)), _full((1, 512)),
            _full((512, NPAT)), _full((1, NPAT)),
            _full((NPAT, 64)), _full((1, 64)),
            _full((64, 1)), _full((1, 1)),
            _full((FEAT + NPAT, FEAT)), _full((1, FEAT)),
            _full((FEAT, 10)), _full((1, 10)),
        ],
        out_specs=[
            pl.BlockSpec((NB, 10), lambda i: (i, 0)),
            pl.BlockSpec((NB, NPAT), lambda i: (i, 0)),
            pl.BlockSpec((NB, 1), lambda i: (i, 0)),
        ],
        out_shape=[
            jax.ShapeDtypeStruct((N, 10), jnp.float32),
            jax.ShapeDtypeStruct((N, NPAT), jnp.float32),
            jax.ShapeDtypeStruct((N, 1), jnp.float32),
        ],
    )(x, w1, b1, w2s, b2, w3, b3, wcb1, bcb1, wcb2, bcb2, wt1, bt1, wt2, bt2)


_q = np.arange(256)
_PERM = np.asarray(8 * (_q // 32) + (_q % 32) // 4, dtype=np.int32)
_REP = np.zeros((64, 256), np.float32)
_REP[_PERM, _q] = 1.0
_TILE = np.zeros((32, 256), np.float32)
_TILE[_q % 32, _q] = 1.0
_SRED = np.zeros((256, 32), np.float32)
_SRED[_q, 4 * (_q // 32) + _q % 4] = 1.0


_qq = np.arange(1024)
_j8 = _qq // 128
_r128 = _qq % 128
_u4 = _r128 // 32
_i8 = (_r128 % 32) // 4
_c4 = _qq % 4
_BREP = np.zeros((256, 1024), np.float32)
_BREP[64 * _u4 + 8 * _i8 + _j8, _qq] = 1.0
_BTILE = np.zeros((128, 1024), np.float32)
_BTILE[32 * _u4 + 4 * _j8 + _c4, _qq] = 1.0


def _np_bd4(w):
    k, m = w.shape
    out = np.zeros((4 * k, 4 * m), np.float32)
    for u in range(4):
        out[u * k:(u + 1) * k, u * m:(u + 1) * m] = w
    return out


def kernel(grid, edge_index, Wf1, bf1, Wf2, bf2, Wmap, bmap, Wl, bl, Wc1,
           bc1, Wc2, bc2, Wc3, bc3, Wcb1, bcb1, Wcb2, bcb2, Wt1, bt1, Wt2,
           bt2):
    pad = EP - E
    ei_g = jnp.concatenate(
        [edge_index, jnp.zeros((2, pad), jnp.int32)], axis=1)
    dst_s = jnp.concatenate(
        [edge_index[1], jnp.full((pad,), N, jnp.int32)])
    zrows = jnp.zeros((SR, FEAT), jnp.float32)

    NE = EP // 2
    g_a, s_a = _get_sc_kernels(NE, 0)
    g_b, s_b = _get_sc_kernels(NE, NE)
    xp = _k1(grid.reshape(N // 4, 40), Wf1, bf1, Wf2, bf2)
    for l in range(2):
        x_lin = xp.reshape(N, FEAT)
        w1l, w2l, bml = Wmap[l][:FEAT], Wmap[l][FEAT:], bmap[l][None]
        xga = g_a(x_lin, ei_g)
        xgb = g_b(x_lin, ei_g)
        msg_a = _k2(xga.reshape(2 * NE // 4, 128), w1l, w2l, bml, NE)
        msg_b = _k2(xgb.reshape(2 * NE // 4, 128), w1l, w2l, bml, NE)
        agg_a = s_a(msg_a.reshape(NE, FEAT), dst_s, zrows)
        agg_b = s_b(msg_b.reshape(NE, FEAT), dst_s, zrows)
        xp = _k3(xp, agg_a.reshape(N // 4, 128), agg_b.reshape(N // 4, 128),
                 Wl[l], bl[l][None])
    x = xp.reshape(N, FEAT)

    w1all = jnp.transpose(Wc1, (1, 0, 2)).reshape(FEAT, NPAT * 64)
    b1all = bc1.reshape(1, NPAT * 64)
    b2all = bc2.reshape(1, NPAT * 64)
    w2pair = jnp.stack([
        jax.scipy.linalg.block_diag(Wc2[2 * p], Wc2[2 * p + 1])
        for p in range(NPAT // 2)
    ])
    w3blk = jax.scipy.linalg.block_diag(*[Wc3[p] for p in range(NPAT)])
    b3all = bc3.reshape(1, NPAT)
    out, individual, combined = _k4(
        x, w1all, b1all, w2pair, b2all, w3blk, b3all, Wcb1, bcb1[None],
        Wcb2, bcb2[None], Wt1, bt1[None], Wt2, bt2[None])
    return out, individual, combined
